# Initial kernel scaffold; baseline (speedup 1.0000x reference)
import jax
import jax.numpy as jnp
from jax.experimental import pallas as pl


def kernel(node_feats, edge_index, edge_w, rel_pos, l0_rw1, l0_rb1, l0_rg1, l0_rbt1, l0_rw2, l0_rb2, l0_rg2, l0_rbt2, l0_w3, l0_wq, l0_wproj, l0_ng, l0_nb, l1_rw1, l1_rb1, l1_rg1, l1_rbt1, l1_rw2, l1_rb2, l1_rg2, l1_rbt2, l1_w3, l1_wq, l1_wproj, l1_ng, l1_nb, f_rw1, f_rb1, f_rg1, f_rbt1, f_rw2, f_rb2, f_rg2, f_rbt2, f_w3, f_wself):
    raise NotImplementedError("write your pallas kernel here")


# speedup vs baseline: 9.4873x; 9.4873x over previous
import functools
import jax
import jax.numpy as jnp
from jax import lax
from jax.experimental import pallas as pl
from jax.experimental.pallas import tpu as pltpu
from jax.experimental.pallas import tpu_sc as plsc

F32 = jnp.float32

N = 10000
E = 160000
DIN = 128
MID = 16
C = 32
CKV = 16
H = 8
DH = 2
OUT = 1280

BE = 800
BN = 400
NW = 32
CH = 128


def _ln(x, g, b):
    mu = jnp.mean(x, axis=-1, keepdims=True)
    var = jnp.mean((x - mu) * (x - mu), axis=-1, keepdims=True)
    return (x - mu) * lax.rsqrt(var + 1e-5) * g + b


def _radial_body(ew, rp, w1, b1, g1, bt1, w2, b2, g2, bt2, h0, h1, hf):
    rpv = rp[...]
    r = jnp.sqrt(jnp.sum(rpv * rpv, axis=1, keepdims=True))
    es = jnp.concatenate([ew[...], r], axis=1)
    outs = (h0, h1, hf)
    for j in range(3):
        x = jnp.dot(es, w1[j], preferred_element_type=F32) + b1[j : j + 1]
        x = jax.nn.relu(_ln(x, g1[j : j + 1], bt1[j : j + 1]))
        x = jnp.dot(x, w2[j], preferred_element_type=F32) + b2[j : j + 1]
        x = jax.nn.relu(_ln(x, g2[j : j + 1], bt2[j : j + 1]))
        outs[j][...] = x


def _radial(ew, rp, w1, b1, g1, bt1, w2, b2, g2, bt2):
    nblk = E // BE
    eb = lambda d: pl.BlockSpec((BE, d), lambda i: (i, 0))
    full = lambda s: pl.BlockSpec(s, lambda i: tuple(0 for _ in s))
    return pl.pallas_call(
        _radial_body,
        grid=(nblk,),
        in_specs=[
            eb(4), eb(3),
            full((3, 5, MID)), full((3, MID)), full((3, MID)), full((3, MID)),
            full((3, MID, MID)), full((3, MID)), full((3, MID)), full((3, MID)),
        ],
        out_specs=[eb(MID), eb(MID), eb(MID)],
        out_shape=[jax.ShapeDtypeStruct((E, MID), F32)] * 3,
    )(ew, rp, w1, b1, g1, bt1, w2, b2, g2, bt2)


def _mm_body(x, w, o):
    o[...] = jnp.dot(x[...], w[...], preferred_element_type=F32)


def _mm(x, w):
    n, k = x.shape
    m = w.shape[1]
    return pl.pallas_call(
        _mm_body,
        grid=(n // BN,),
        in_specs=[
            pl.BlockSpec((BN, k), lambda i: (i, 0)),
            pl.BlockSpec((k, m), lambda i: (0, 0)),
        ],
        out_specs=pl.BlockSpec((BN, m), lambda i: (i, 0)),
        out_shape=jax.ShapeDtypeStruct((n, m), F32),
    )(x, w)


def _edgec_body(h, fs, qd, w3all, sel1, summ, pair, lg, v, gmax):
    t = jnp.dot(fs[...], w3all[...], preferred_element_type=F32)
    kvw = t * jnp.dot(h[...], sel1[...], preferred_element_type=F32)
    kv = jnp.dot(kvw, summ[...], preferred_element_type=F32)
    k = kv[:, :CKV]
    vv = kv[:, CKV:]
    prod = k * qd[...]
    lgb = jnp.dot(prod, pair[...], preferred_element_type=F32) * (DH ** -0.5)
    lg[...] = lgb
    v[...] = vv
    bm = jnp.max(lgb, axis=0, keepdims=True)
    i = pl.program_id(0)

    @pl.when(i == 0)
    def _():
        gmax[...] = bm

    @pl.when(i > 0)
    def _():
        gmax[...] = jnp.maximum(gmax[...], bm)


def _edgec(h, fs, qd, w3all, sel1, summ, pair):
    din = fs.shape[1]
    nblk = E // BE
    eb = lambda d: pl.BlockSpec((BE, d), lambda i: (i, 0))
    full = lambda s: pl.BlockSpec(s, lambda i: (0, 0))
    return pl.pallas_call(
        _edgec_body,
        grid=(nblk,),
        in_specs=[
            eb(MID), eb(din), eb(CKV),
            full((din, 2 * CKV * MID)), full((MID, 2 * CKV * MID)),
            full((2 * CKV * MID, 2 * CKV)), full((CKV, H)),
        ],
        out_specs=[eb(H), eb(CKV), full((1, H))],
        out_shape=[
            jax.ShapeDtypeStruct((E, H), F32),
            jax.ShapeDtypeStruct((E, CKV), F32),
            jax.ShapeDtypeStruct((1, H), F32),
        ],
    )(h, fs, qd, w3all, sel1, summ, pair)


def _edged_body(lg, v, gmax, expand, scv):
    ex = jnp.exp(lg[...] - gmax[...])
    num = jnp.dot(ex, expand[...], preferred_element_type=F32) * v[...]
    z8 = jnp.zeros((BE, H), F32)
    scv[...] = jnp.concatenate([ex, num, z8], axis=1)


def _edged(lg, v, gmax, expand):
    nblk = E // BE
    eb = lambda d: pl.BlockSpec((BE, d), lambda i: (i, 0))
    full = lambda s: pl.BlockSpec(s, lambda i: (0, 0))
    return pl.pallas_call(
        _edged_body,
        grid=(nblk,),
        in_specs=[eb(H), eb(CKV), full((1, H)), full((H, CKV))],
        out_specs=eb(C),
        out_shape=jax.ShapeDtypeStruct((E, C), F32),
    )(lg, v, gmax, expand)


def _nodee_body(parts, f, wpz, wpf, ng, nb, expand, wqn, fout, qout):
    p = parts[...]
    s = p[0] + p[1]
    den = s[:, :H]
    num = s[:, H : H + CKV]
    dexp = jnp.dot(den, expand[...], preferred_element_type=F32) + 1e-30
    z = num / dexp
    fp = jnp.dot(z, wpz[...], preferred_element_type=F32) + jnp.dot(
        f[...], wpf[...], preferred_element_type=F32)
    nf = jnp.abs(fp)
    nn = jax.nn.relu(_ln(nf, ng[...], nb[...]))
    fnew = fp / (nf + 1e-8) * nn
    fout[...] = fnew
    if qout is not None:
        qout[...] = jnp.dot(fnew, wqn[...], preferred_element_type=F32)


def _nodee(parts, f, wpz, wpf, ng, nb, expand, wqn):
    din = f.shape[1]
    full = lambda s: pl.BlockSpec(s, lambda i: tuple(0 for _ in s))
    in_specs = [
        pl.BlockSpec((2, BN, C), lambda i: (0, i, 0)),
        pl.BlockSpec((BN, din), lambda i: (i, 0)),
        full((CKV, C)), full((din, C)), full((1, C)), full((1, C)),
        full((H, CKV)),
    ]
    out_specs = [pl.BlockSpec((BN, C), lambda i: (i, 0))]
    out_shape = [jax.ShapeDtypeStruct((N, C), F32)]
    if wqn is not None:
        in_specs.append(full((C, CKV)))
        out_specs.append(pl.BlockSpec((BN, CKV), lambda i: (i, 0)))
        out_shape.append(jax.ShapeDtypeStruct((N, CKV), F32))
        body = _nodee_body
        args = (parts, f, wpz, wpf, ng, nb, expand, wqn)
    else:
        def body(parts, f, wpz, wpf, ng, nb, expand, fout):
            _nodee_body(parts, f, wpz, wpf, ng, nb, expand, None, fout, None)
        args = (parts, f, wpz, wpf, ng, nb, expand)
    return pl.pallas_call(
        body,
        grid=(N // BN,),
        in_specs=in_specs,
        out_specs=out_specs,
        out_shape=out_shape,
    )(*args)


def _outer_body(h, fs, selh, self_, g):
    g[...] = jnp.dot(h[...], selh[...], preferred_element_type=F32) * jnp.dot(
        fs[...], self_[...], preferred_element_type=F32)


def _outer(h, fs, selh, self_):
    nblk = E // BE
    eb = lambda d: pl.BlockSpec((BE, d), lambda i: (i, 0))
    full = lambda s: pl.BlockSpec(s, lambda i: (0, 0))
    return pl.pallas_call(
        _outer_body,
        grid=(nblk,),
        in_specs=[eb(MID), eb(C), full((MID, MID * C)), full((C, MID * C))],
        out_specs=eb(MID * C),
        out_shape=jax.ShapeDtypeStruct((E, MID * C), F32),
    )(h, fs, selh, self_)


def _final_body(parts, f, wf, wself, out, pooled):
    p = parts[...]
    a = p[0] + p[1]
    o = jnp.dot(a, wf[...], preferred_element_type=F32) + jnp.dot(
        f[...], wself[...], preferred_element_type=F32)
    out[...] = o
    bm = jnp.max(o, axis=0, keepdims=True)
    i = pl.program_id(0)

    @pl.when(i == 0)
    def _():
        pooled[...] = bm

    @pl.when(i > 0)
    def _():
        pooled[...] = jnp.maximum(pooled[...], bm)


def _final(parts, f, wf, wself):
    full = lambda s: pl.BlockSpec(s, lambda i: (0, 0))
    return pl.pallas_call(
        _final_body,
        grid=(N // BN,),
        in_specs=[
            pl.BlockSpec((2, BN, MID * C), lambda i: (0, i, 0)),
            pl.BlockSpec((BN, C), lambda i: (i, 0)),
            full((MID * C, OUT)), full((C, OUT)),
        ],
        out_specs=[pl.BlockSpec((BN, OUT), lambda i: (i, 0)), full((1, OUT))],
        out_shape=[
            jax.ShapeDtypeStruct((N, OUT), F32),
            jax.ShapeDtypeStruct((1, OUT), F32),
        ],
    )(parts, f, wf, wself)


def _sc_gather(table, idx):
    d = table.shape[1]
    e = idx.shape[0]
    nch = e // CH
    iters = (nch + NW - 1) // NW
    mesh = plsc.VectorSubcoreMesh(core_axis_name="c", subcore_axis_name="s")

    @functools.partial(
        pl.kernel,
        out_type=jax.ShapeDtypeStruct((e, d), F32),
        mesh=mesh,
        compiler_params=pltpu.CompilerParams(use_tc_tiling_on_sc=False),
        scratch_types=[
            pltpu.VMEM((CH,), jnp.int32),
            pltpu.VMEM((CH, d), F32),
            pltpu.SemaphoreType.DMA,
        ],
    )
    def k(table_hbm, idx_hbm, out_hbm, idx_v, rows_v, sem):
        wid = lax.axis_index("s") * 2 + lax.axis_index("c")

        def body(c, _):
            chunk = wid + NW * c

            @pl.when(chunk < nch)
            def _():
                base = chunk * CH
                pltpu.sync_copy(idx_hbm.at[pl.ds(base, CH)], idx_v)
                pltpu.async_copy(table_hbm.at[idx_v], rows_v, sem).wait()
                pltpu.sync_copy(rows_v, out_hbm.at[pl.ds(base, CH)])

            return 0

        lax.fori_loop(0, iters, body, 0)

    return k(table, idx)


def _sc_scatter_add(vals, dst, fc, fchunks):
    e = dst.shape[0]
    half = e // 2
    nch = half // CH
    iters = (nch + 15) // 16
    rz = N // 16
    zc = 25
    ftot = vals.shape[1]
    mesh = plsc.VectorSubcoreMesh(core_axis_name="c", subcore_axis_name="s")

    @functools.partial(
        pl.kernel,
        out_type=jax.ShapeDtypeStruct((2, N, ftot), F32),
        mesh=mesh,
        compiler_params=pltpu.CompilerParams(use_tc_tiling_on_sc=False),
        scratch_types=[
            pltpu.VMEM((CH,), jnp.int32),
            pltpu.VMEM((CH, fc), F32),
            pltpu.VMEM((zc, fc), F32),
            pltpu.VMEM_SHARED((N, fc), F32),
            pltpu.SemaphoreType.DMA,
        ],
    )
    def k(vals_hbm, dst_hbm, out_hbm, idx_v, v_v, zbuf, acc, sem):
        cid = lax.axis_index("c")
        sid = lax.axis_index("s")
        for rr in range(zc):
            for j in range(fc // 16):
                zbuf[rr, pl.ds(j * 16, 16)] = jnp.zeros((16,), F32)
        for p in range(fchunks):
            fo = p * fc

            def zbody(cz, _):
                r = sid * rz + cz * zc
                pltpu.sync_copy(zbuf, acc.at[pl.ds(r, zc)])
                return 0

            lax.fori_loop(0, rz // zc, zbody, 0)
            plsc.subcore_barrier()

            def body(cc, _):
                chunk = sid + 16 * cc

                @pl.when(chunk < nch)
                def _():
                    base = cid * half + chunk * CH
                    pltpu.sync_copy(dst_hbm.at[pl.ds(base, CH)], idx_v)
                    pltpu.sync_copy(
                        vals_hbm.at[pl.ds(base, CH), pl.ds(fo, fc)], v_v)
                    pltpu.sync_copy(v_v, acc.at[idx_v], add=True)

                return 0

            lax.fori_loop(0, iters, body, 0)
            plsc.subcore_barrier()

            def wbody(cw, _):
                r = sid * rz + cw * zc
                pltpu.sync_copy(
                    acc.at[pl.ds(r, zc)],
                    out_hbm.at[cid, pl.ds(r, zc), pl.ds(fo, fc)])
                return 0

            lax.fori_loop(0, rz // zc, wbody, 0)
            plsc.subcore_barrier()

    return k(vals, dst)


def _layer(f, q, src, dst, h, w3, wproj, ng, nb, wq_next, consts):
    sel1, summ, pair, expand = consts
    din = f.shape[1]
    fs = _sc_gather(f, src)
    qd = _sc_gather(q, dst)
    w3all = w3.transpose(2, 0, 1).reshape(din, MID * 2 * CKV)
    lg, v, gmax = _edgec(h, fs, qd, w3all, sel1, summ, pair)
    scv = _edged(lg, v, gmax, expand)
    parts = _sc_scatter_add(scv, dst, C, 1)
    wpz = wproj[:CKV]
    wpf = wproj[CKV:]
    return _nodee(parts, f, wpz, wpf, ng.reshape(1, C), nb.reshape(1, C),
                  expand, wq_next)


def kernel(node_feats, edge_index, edge_w, rel_pos,
           l0_rw1, l0_rb1, l0_rg1, l0_rbt1, l0_rw2, l0_rb2, l0_rg2, l0_rbt2,
           l0_w3, l0_wq, l0_wproj, l0_ng, l0_nb,
           l1_rw1, l1_rb1, l1_rg1, l1_rbt1, l1_rw2, l1_rb2, l1_rg2, l1_rbt2,
           l1_w3, l1_wq, l1_wproj, l1_ng, l1_nb,
           f_rw1, f_rb1, f_rg1, f_rbt1, f_rw2, f_rb2, f_rg2, f_rbt2,
           f_w3, f_wself):
    src = edge_index[0]
    dst = edge_index[1]

    w1 = jnp.stack([l0_rw1, l1_rw1, f_rw1])
    b1 = jnp.stack([l0_rb1, l1_rb1, f_rb1])
    g1 = jnp.stack([l0_rg1, l1_rg1, f_rg1])
    bt1 = jnp.stack([l0_rbt1, l1_rbt1, f_rbt1])
    w2 = jnp.stack([l0_rw2, l1_rw2, f_rw2])
    b2 = jnp.stack([l0_rb2, l1_rb2, f_rb2])
    g2 = jnp.stack([l0_rg2, l1_rg2, f_rg2])
    bt2 = jnp.stack([l0_rbt2, l1_rbt2, f_rbt2])
    h0, h1, hf = _radial(edge_w, rel_pos, w1, b1, g1, bt1, w2, b2, g2, bt2)

    sel1 = jnp.kron(jnp.eye(MID, dtype=F32), jnp.ones((1, 2 * CKV), F32))
    summ = jnp.tile(jnp.eye(2 * CKV, dtype=F32), (MID, 1))
    pair = jnp.kron(jnp.eye(H, dtype=F32), jnp.ones((DH, 1), F32))
    expand = jnp.kron(jnp.eye(H, dtype=F32), jnp.ones((1, DH), F32))
    consts = (sel1, summ, pair, expand)

    q0 = _mm(node_feats, l0_wq)
    f1, q1 = _layer(node_feats, q0, src, dst, h0, l0_w3, l0_wproj,
                    l0_ng, l0_nb, l1_wq, consts)
    f2 = _layer(f1, q1, src, dst, h1, l1_w3, l1_wproj,
                l1_ng, l1_nb, None, consts)[0]

    fsf = _sc_gather(f2, src)
    selh = jnp.kron(jnp.eye(MID, dtype=F32), jnp.ones((1, C), F32))
    self_ = jnp.tile(jnp.eye(C, dtype=F32), (1, MID))
    g = _outer(hf, fsf, selh, self_)
    partsf = _sc_scatter_add(g, dst, 128, 4)
    wf = f_w3.transpose(0, 2, 1).reshape(MID * C, OUT)
    out, pooled = _final(partsf, f2, wf, f_wself)
    return (out[:, :, None], pooled)


# speedup vs baseline: 10.5659x; 1.1137x over previous
import functools
import jax
import jax.numpy as jnp
from jax import lax
from jax.experimental import pallas as pl
from jax.experimental.pallas import tpu as pltpu
from jax.experimental.pallas import tpu_sc as plsc

F32 = jnp.float32

N = 10000
E = 160000
DIN = 128
MID = 16
C = 32
CKV = 16
H = 8
DH = 2
OUT = 1280

BE = 800
BN = 400
NW = 32
CH = 128


def _ln(x, g, b):
    mu = jnp.mean(x, axis=-1, keepdims=True)
    var = jnp.mean((x - mu) * (x - mu), axis=-1, keepdims=True)
    return (x - mu) * lax.rsqrt(var + 1e-5) * g + b


def _radial_body(ew, rp, w1, b1, g1, bt1, w2, b2, g2, bt2, h0, h1, hf):
    rpv = rp[...]
    r = jnp.sqrt(jnp.sum(rpv * rpv, axis=1, keepdims=True))
    es = jnp.concatenate([ew[...], r], axis=1)
    outs = (h0, h1, hf)
    for j in range(3):
        x = jnp.dot(es, w1[j], preferred_element_type=F32) + b1[j : j + 1]
        x = jax.nn.relu(_ln(x, g1[j : j + 1], bt1[j : j + 1]))
        x = jnp.dot(x, w2[j], preferred_element_type=F32) + b2[j : j + 1]
        x = jax.nn.relu(_ln(x, g2[j : j + 1], bt2[j : j + 1]))
        outs[j][...] = x


def _radial(ew, rp, w1, b1, g1, bt1, w2, b2, g2, bt2):
    nblk = E // BE
    eb = lambda d: pl.BlockSpec((BE, d), lambda i: (i, 0))
    full = lambda s: pl.BlockSpec(s, lambda i: tuple(0 for _ in s))
    return pl.pallas_call(
        _radial_body,
        grid=(nblk,),
        in_specs=[
            eb(4), eb(3),
            full((3, 5, MID)), full((3, MID)), full((3, MID)), full((3, MID)),
            full((3, MID, MID)), full((3, MID)), full((3, MID)), full((3, MID)),
        ],
        out_specs=[eb(MID), eb(MID), eb(MID)],
        out_shape=[jax.ShapeDtypeStruct((E, MID), F32)] * 3,
    )(ew, rp, w1, b1, g1, bt1, w2, b2, g2, bt2)


def _mm_body(x, w, o):
    o[...] = jnp.dot(x[...], w[...], preferred_element_type=F32)


def _mm(x, w):
    n, k = x.shape
    m = w.shape[1]
    return pl.pallas_call(
        _mm_body,
        grid=(n // BN,),
        in_specs=[
            pl.BlockSpec((BN, k), lambda i: (i, 0)),
            pl.BlockSpec((k, m), lambda i: (0, 0)),
        ],
        out_specs=pl.BlockSpec((BN, m), lambda i: (i, 0)),
        out_shape=jax.ShapeDtypeStruct((n, m), F32),
    )(x, w)


def _edgec_body(h, fs, qd, w3all, sel1, summ, pair, lg, v, gmax):
    t = jnp.dot(fs[...], w3all[...], preferred_element_type=F32)
    kvw = t * jnp.dot(h[...], sel1[...], preferred_element_type=F32)
    kv = jnp.dot(kvw, summ[...], preferred_element_type=F32)
    k = kv[:, :CKV]
    vv = kv[:, CKV:]
    prod = k * qd[...]
    lgb = jnp.dot(prod, pair[...], preferred_element_type=F32) * (DH ** -0.5)
    lg[...] = lgb
    v[...] = vv
    bm = jnp.max(lgb, axis=0, keepdims=True)
    i = pl.program_id(0)

    @pl.when(i == 0)
    def _():
        gmax[...] = bm

    @pl.when(i > 0)
    def _():
        gmax[...] = jnp.maximum(gmax[...], bm)


def _edgec(h, fs, qd, w3all, sel1, summ, pair):
    din = fs.shape[1]
    nblk = E // BE
    eb = lambda d: pl.BlockSpec((BE, d), lambda i: (i, 0))
    full = lambda s: pl.BlockSpec(s, lambda i: (0, 0))
    return pl.pallas_call(
        _edgec_body,
        grid=(nblk,),
        in_specs=[
            eb(MID), eb(din), eb(CKV),
            full((din, 2 * CKV * MID)), full((MID, 2 * CKV * MID)),
            full((2 * CKV * MID, 2 * CKV)), full((CKV, H)),
        ],
        out_specs=[eb(H), eb(CKV), full((1, H))],
        out_shape=[
            jax.ShapeDtypeStruct((E, H), F32),
            jax.ShapeDtypeStruct((E, CKV), F32),
            jax.ShapeDtypeStruct((1, H), F32),
        ],
    )(h, fs, qd, w3all, sel1, summ, pair)


def _edged_body(lg, v, gmax, expand, scv):
    ex = jnp.exp(lg[...] - gmax[...])
    num = jnp.dot(ex, expand[...], preferred_element_type=F32) * v[...]
    z8 = jnp.zeros((BE, H), F32)
    scv[...] = jnp.concatenate([ex, num, z8], axis=1)


def _edged(lg, v, gmax, expand):
    nblk = E // BE
    eb = lambda d: pl.BlockSpec((BE, d), lambda i: (i, 0))
    full = lambda s: pl.BlockSpec(s, lambda i: (0, 0))
    return pl.pallas_call(
        _edged_body,
        grid=(nblk,),
        in_specs=[eb(H), eb(CKV), full((1, H)), full((H, CKV))],
        out_specs=eb(C),
        out_shape=jax.ShapeDtypeStruct((E, C), F32),
    )(lg, v, gmax, expand)


def _nodee_body(parts, f, wpz, wpf, ng, nb, expand, wqn, fout, qout):
    p = parts[...]
    s = p[0] + p[1]
    den = s[:, :H]
    num = s[:, H : H + CKV]
    dexp = jnp.dot(den, expand[...], preferred_element_type=F32) + 1e-30
    z = num / dexp
    fp = jnp.dot(z, wpz[...], preferred_element_type=F32) + jnp.dot(
        f[...], wpf[...], preferred_element_type=F32)
    nf = jnp.abs(fp)
    nn = jax.nn.relu(_ln(nf, ng[...], nb[...]))
    fnew = fp / (nf + 1e-8) * nn
    fout[...] = fnew
    if qout is not None:
        qout[...] = jnp.dot(fnew, wqn[...], preferred_element_type=F32)


def _nodee(parts, f, wpz, wpf, ng, nb, expand, wqn):
    din = f.shape[1]
    full = lambda s: pl.BlockSpec(s, lambda i: tuple(0 for _ in s))
    in_specs = [
        pl.BlockSpec((2, BN, C), lambda i: (0, i, 0)),
        pl.BlockSpec((BN, din), lambda i: (i, 0)),
        full((CKV, C)), full((din, C)), full((1, C)), full((1, C)),
        full((H, CKV)),
    ]
    out_specs = [pl.BlockSpec((BN, C), lambda i: (i, 0))]
    out_shape = [jax.ShapeDtypeStruct((N, C), F32)]
    if wqn is not None:
        in_specs.append(full((C, CKV)))
        out_specs.append(pl.BlockSpec((BN, CKV), lambda i: (i, 0)))
        out_shape.append(jax.ShapeDtypeStruct((N, CKV), F32))
        body = _nodee_body
        args = (parts, f, wpz, wpf, ng, nb, expand, wqn)
    else:
        def body(parts, f, wpz, wpf, ng, nb, expand, fout):
            _nodee_body(parts, f, wpz, wpf, ng, nb, expand, None, fout, None)
        args = (parts, f, wpz, wpf, ng, nb, expand)
    return pl.pallas_call(
        body,
        grid=(N // BN,),
        in_specs=in_specs,
        out_specs=out_specs,
        out_shape=out_shape,
    )(*args)


def _outer_body(h, fs, selh, self_, g):
    g[...] = jnp.dot(h[...], selh[...], preferred_element_type=F32) * jnp.dot(
        fs[...], self_[...], preferred_element_type=F32)


def _outer(h, fs, selh, self_):
    nblk = E // BE
    eb = lambda d: pl.BlockSpec((BE, d), lambda i: (i, 0))
    full = lambda s: pl.BlockSpec(s, lambda i: (0, 0))
    return pl.pallas_call(
        _outer_body,
        grid=(nblk,),
        in_specs=[eb(MID), eb(C), full((MID, MID * C)), full((C, MID * C))],
        out_specs=eb(MID * C),
        out_shape=jax.ShapeDtypeStruct((E, MID * C), F32),
    )(h, fs, selh, self_)


def _final_body(parts, f, wf, wself, out, pooled):
    p = parts[...]
    a = p[0] + p[1]
    o = jnp.dot(a, wf[...], preferred_element_type=F32) + jnp.dot(
        f[...], wself[...], preferred_element_type=F32)
    out[...] = o
    bm = jnp.max(o, axis=0, keepdims=True)
    i = pl.program_id(0)

    @pl.when(i == 0)
    def _():
        pooled[...] = bm

    @pl.when(i > 0)
    def _():
        pooled[...] = jnp.maximum(pooled[...], bm)


def _final(parts, f, wf, wself):
    full = lambda s: pl.BlockSpec(s, lambda i: (0, 0))
    return pl.pallas_call(
        _final_body,
        grid=(N // BN,),
        in_specs=[
            pl.BlockSpec((2, BN, MID * C), lambda i: (0, i, 0)),
            pl.BlockSpec((BN, C), lambda i: (i, 0)),
            full((MID * C, OUT)), full((C, OUT)),
        ],
        out_specs=[pl.BlockSpec((BN, OUT), lambda i: (i, 0)), full((1, OUT))],
        out_shape=[
            jax.ShapeDtypeStruct((N, OUT), F32),
            jax.ShapeDtypeStruct((1, OUT), F32),
        ],
    )(parts, f, wf, wself)


def _sc_gather(table, idx):
    d = table.shape[1]
    e = idx.shape[0]
    nch = e // CH
    iters = (nch + NW - 1) // NW
    tiled = d % 128 == 0
    mesh = plsc.VectorSubcoreMesh(core_axis_name="c", subcore_axis_name="s")

    @functools.partial(
        pl.kernel,
        out_type=jax.ShapeDtypeStruct((e, d), F32),
        mesh=mesh,
        compiler_params=pltpu.CompilerParams(use_tc_tiling_on_sc=tiled),
        scratch_types=[
            pltpu.VMEM((CH,), jnp.int32),
            pltpu.VMEM((CH, d), F32),
            pltpu.SemaphoreType.DMA,
        ],
    )
    def k(table_hbm, idx_hbm, out_hbm, idx_v, rows_v, sem):
        wid = lax.axis_index("s") * 2 + lax.axis_index("c")

        def body(c, _):
            chunk = wid + NW * c

            @pl.when(chunk < nch)
            def _():
                base = chunk * CH
                pltpu.sync_copy(idx_hbm.at[pl.ds(base, CH)], idx_v)
                pltpu.async_copy(table_hbm.at[idx_v], rows_v, sem).wait()
                pltpu.sync_copy(rows_v, out_hbm.at[pl.ds(base, CH)])

            return 0

        lax.fori_loop(0, iters, body, 0)

    return k(table, idx)


def _sc_scatter_add(vals, dst, fc, fchunks):
    e = dst.shape[0]
    half = e // 2
    nch = half // CH
    iters = (nch + 15) // 16
    zc = 40
    nrc = N // zc
    riters = (nrc + 15) // 16
    ftot = vals.shape[1]
    tiled = fc % 128 == 0
    mesh = plsc.VectorSubcoreMesh(core_axis_name="c", subcore_axis_name="s")

    @functools.partial(
        pl.kernel,
        out_type=jax.ShapeDtypeStruct((2, N, ftot), F32),
        mesh=mesh,
        compiler_params=pltpu.CompilerParams(use_tc_tiling_on_sc=tiled),
        scratch_types=[
            pltpu.VMEM((CH,), jnp.int32),
            pltpu.VMEM((CH, fc), F32),
            pltpu.VMEM((zc, fc), F32),
            pltpu.VMEM_SHARED((N, fc), F32),
            pltpu.SemaphoreType.DMA,
        ],
    )
    def k(vals_hbm, dst_hbm, out_hbm, idx_v, v_v, zbuf, acc, sem):
        cid = lax.axis_index("c")
        sid = lax.axis_index("s")
        for rr in range(zc):
            for j in range(fc // 16):
                zbuf[rr, pl.ds(j * 16, 16)] = jnp.zeros((16,), F32)
        for p in range(fchunks):
            fo = p * fc

            def zbody(cz, _):
                rc = sid + 16 * cz

                @pl.when(rc < nrc)
                def _():
                    pltpu.sync_copy(zbuf, acc.at[pl.ds(rc * zc, zc)])

                return 0

            lax.fori_loop(0, riters, zbody, 0)
            plsc.subcore_barrier()

            def body(cc, _):
                chunk = sid + 16 * cc

                @pl.when(chunk < nch)
                def _():
                    base = cid * half + chunk * CH
                    pltpu.sync_copy(dst_hbm.at[pl.ds(base, CH)], idx_v)
                    pltpu.sync_copy(
                        vals_hbm.at[pl.ds(base, CH), pl.ds(fo, fc)], v_v)
                    pltpu.sync_copy(v_v, acc.at[idx_v], add=True)

                return 0

            lax.fori_loop(0, iters, body, 0)
            plsc.subcore_barrier()

            def wbody(cw, _):
                rc = sid + 16 * cw

                @pl.when(rc < nrc)
                def _():
                    pltpu.sync_copy(
                        acc.at[pl.ds(rc * zc, zc)],
                        out_hbm.at[cid, pl.ds(rc * zc, zc), pl.ds(fo, fc)])

                return 0

            lax.fori_loop(0, riters, wbody, 0)
            plsc.subcore_barrier()

    return k(vals, dst)


def _layer(f, q, src, dst, h, w3, wproj, ng, nb, wq_next, consts):
    sel1, summ, pair, expand = consts
    din = f.shape[1]
    fs = _sc_gather(f, src)
    qd = _sc_gather(q, dst)
    w3all = w3.transpose(2, 0, 1).reshape(din, MID * 2 * CKV)
    lg, v, gmax = _edgec(h, fs, qd, w3all, sel1, summ, pair)
    scv = _edged(lg, v, gmax, expand)
    parts = _sc_scatter_add(scv, dst, C, 1)
    wpz = wproj[:CKV]
    wpf = wproj[CKV:]
    return _nodee(parts, f, wpz, wpf, ng.reshape(1, C), nb.reshape(1, C),
                  expand, wq_next)


def kernel(node_feats, edge_index, edge_w, rel_pos,
           l0_rw1, l0_rb1, l0_rg1, l0_rbt1, l0_rw2, l0_rb2, l0_rg2, l0_rbt2,
           l0_w3, l0_wq, l0_wproj, l0_ng, l0_nb,
           l1_rw1, l1_rb1, l1_rg1, l1_rbt1, l1_rw2, l1_rb2, l1_rg2, l1_rbt2,
           l1_w3, l1_wq, l1_wproj, l1_ng, l1_nb,
           f_rw1, f_rb1, f_rg1, f_rbt1, f_rw2, f_rb2, f_rg2, f_rbt2,
           f_w3, f_wself):
    src = edge_index[0]
    dst = edge_index[1]

    w1 = jnp.stack([l0_rw1, l1_rw1, f_rw1])
    b1 = jnp.stack([l0_rb1, l1_rb1, f_rb1])
    g1 = jnp.stack([l0_rg1, l1_rg1, f_rg1])
    bt1 = jnp.stack([l0_rbt1, l1_rbt1, f_rbt1])
    w2 = jnp.stack([l0_rw2, l1_rw2, f_rw2])
    b2 = jnp.stack([l0_rb2, l1_rb2, f_rb2])
    g2 = jnp.stack([l0_rg2, l1_rg2, f_rg2])
    bt2 = jnp.stack([l0_rbt2, l1_rbt2, f_rbt2])
    h0, h1, hf = _radial(edge_w, rel_pos, w1, b1, g1, bt1, w2, b2, g2, bt2)

    sel1 = jnp.kron(jnp.eye(MID, dtype=F32), jnp.ones((1, 2 * CKV), F32))
    summ = jnp.tile(jnp.eye(2 * CKV, dtype=F32), (MID, 1))
    pair = jnp.kron(jnp.eye(H, dtype=F32), jnp.ones((DH, 1), F32))
    expand = jnp.kron(jnp.eye(H, dtype=F32), jnp.ones((1, DH), F32))
    consts = (sel1, summ, pair, expand)

    q0 = _mm(node_feats, l0_wq)
    f1, q1 = _layer(node_feats, q0, src, dst, h0, l0_w3, l0_wproj,
                    l0_ng, l0_nb, l1_wq, consts)
    f2 = _layer(f1, q1, src, dst, h1, l1_w3, l1_wproj,
                l1_ng, l1_nb, None, consts)[0]

    fsf = _sc_gather(f2, src)
    selh = jnp.kron(jnp.eye(MID, dtype=F32), jnp.ones((1, C), F32))
    self_ = jnp.tile(jnp.eye(C, dtype=F32), (1, MID))
    g = _outer(hf, fsf, selh, self_)
    partsf = _sc_scatter_add(g, dst, 128, 4)
    wf = f_w3.transpose(0, 2, 1).reshape(MID * C, OUT)
    out, pooled = _final(partsf, f2, wf, f_wself)
    return (out[:, :, None], pooled)


# speedup vs baseline: 11.4561x; 1.0842x over previous
import functools
import jax
import jax.numpy as jnp
from jax import lax
from jax.experimental import pallas as pl
from jax.experimental.pallas import tpu as pltpu
from jax.experimental.pallas import tpu_sc as plsc

F32 = jnp.float32

N = 10000
E = 160000
DIN = 128
MID = 16
C = 32
CKV = 16
H = 8
DH = 2
OUT = 1280

BE = 800
BN = 400
NW = 32
CH = 128


def _ln(x, g, b):
    mu = jnp.mean(x, axis=-1, keepdims=True)
    var = jnp.mean((x - mu) * (x - mu), axis=-1, keepdims=True)
    return (x - mu) * lax.rsqrt(var + 1e-5) * g + b


def _radial_body(ew, rp, w1, b1, g1, bt1, w2, b2, g2, bt2, h0, h1, hf):
    rpv = rp[...]
    r = jnp.sqrt(jnp.sum(rpv * rpv, axis=1, keepdims=True))
    es = jnp.concatenate([ew[...], r], axis=1)
    outs = (h0, h1, hf)
    for j in range(3):
        x = jnp.dot(es, w1[j], preferred_element_type=F32) + b1[j : j + 1]
        x = jax.nn.relu(_ln(x, g1[j : j + 1], bt1[j : j + 1]))
        x = jnp.dot(x, w2[j], preferred_element_type=F32) + b2[j : j + 1]
        x = jax.nn.relu(_ln(x, g2[j : j + 1], bt2[j : j + 1]))
        outs[j][...] = x


def _radial(ew, rp, w1, b1, g1, bt1, w2, b2, g2, bt2):
    nblk = E // BE
    eb = lambda d: pl.BlockSpec((BE, d), lambda i: (i, 0))
    full = lambda s: pl.BlockSpec(s, lambda i: tuple(0 for _ in s))
    return pl.pallas_call(
        _radial_body,
        grid=(nblk,),
        in_specs=[
            eb(4), eb(3),
            full((3, 5, MID)), full((3, MID)), full((3, MID)), full((3, MID)),
            full((3, MID, MID)), full((3, MID)), full((3, MID)), full((3, MID)),
        ],
        out_specs=[eb(MID), eb(MID), eb(MID)],
        out_shape=[jax.ShapeDtypeStruct((E, MID), F32)] * 3,
    )(ew, rp, w1, b1, g1, bt1, w2, b2, g2, bt2)


def _mm_body(x, w, o):
    o[...] = jnp.dot(x[...], w[...], preferred_element_type=F32)


def _mm(x, w):
    n, k = x.shape
    m = w.shape[1]
    return pl.pallas_call(
        _mm_body,
        grid=(n // BN,),
        in_specs=[
            pl.BlockSpec((BN, k), lambda i: (i, 0)),
            pl.BlockSpec((k, m), lambda i: (0, 0)),
        ],
        out_specs=pl.BlockSpec((BN, m), lambda i: (i, 0)),
        out_shape=jax.ShapeDtypeStruct((n, m), F32),
    )(x, w)


def _edgec_body(h, fs, qd, w3all, sel1, summ, pair, lg, v, gmax):
    t = jnp.dot(fs[...], w3all[...], preferred_element_type=F32)
    kvw = t * jnp.dot(h[...], sel1[...], preferred_element_type=F32)
    kv = jnp.dot(kvw, summ[...], preferred_element_type=F32)
    k = kv[:, :CKV]
    vv = kv[:, CKV:]
    prod = k * qd[...]
    lgb = jnp.dot(prod, pair[...], preferred_element_type=F32) * (DH ** -0.5)
    lg[...] = lgb
    v[...] = vv
    bm = jnp.max(lgb, axis=0, keepdims=True)
    i = pl.program_id(0)

    @pl.when(i == 0)
    def _():
        gmax[...] = bm

    @pl.when(i > 0)
    def _():
        gmax[...] = jnp.maximum(gmax[...], bm)


def _edgec(h, fs, qd, w3all, sel1, summ, pair):
    din = fs.shape[1]
    nblk = E // BE
    eb = lambda d: pl.BlockSpec((BE, d), lambda i: (i, 0))
    full = lambda s: pl.BlockSpec(s, lambda i: (0, 0))
    return pl.pallas_call(
        _edgec_body,
        grid=(nblk,),
        in_specs=[
            eb(MID), eb(din), eb(CKV),
            full((din, 2 * CKV * MID)), full((MID, 2 * CKV * MID)),
            full((2 * CKV * MID, 2 * CKV)), full((CKV, H)),
        ],
        out_specs=[eb(H), eb(CKV), full((1, H))],
        out_shape=[
            jax.ShapeDtypeStruct((E, H), F32),
            jax.ShapeDtypeStruct((E, CKV), F32),
            jax.ShapeDtypeStruct((1, H), F32),
        ],
    )(h, fs, qd, w3all, sel1, summ, pair)


def _edged_body(lg, v, gmax, expand, scv):
    ex = jnp.exp(lg[...] - gmax[...])
    num = jnp.dot(ex, expand[...], preferred_element_type=F32) * v[...]
    z8 = jnp.zeros((BE, H), F32)
    scv[...] = jnp.concatenate([ex, num, z8], axis=1)


def _edged(lg, v, gmax, expand):
    nblk = E // BE
    eb = lambda d: pl.BlockSpec((BE, d), lambda i: (i, 0))
    full = lambda s: pl.BlockSpec(s, lambda i: (0, 0))
    return pl.pallas_call(
        _edged_body,
        grid=(nblk,),
        in_specs=[eb(H), eb(CKV), full((1, H)), full((H, CKV))],
        out_specs=eb(C),
        out_shape=jax.ShapeDtypeStruct((E, C), F32),
    )(lg, v, gmax, expand)


def _nodee_body(parts, f, wpz, wpf, ng, nb, expand, wqn, fout, qout):
    p = parts[...]
    s = p[0] + p[1]
    den = s[:, :H]
    num = s[:, H : H + CKV]
    dexp = jnp.dot(den, expand[...], preferred_element_type=F32) + 1e-30
    z = num / dexp
    fp = jnp.dot(z, wpz[...], preferred_element_type=F32) + jnp.dot(
        f[...], wpf[...], preferred_element_type=F32)
    nf = jnp.abs(fp)
    nn = jax.nn.relu(_ln(nf, ng[...], nb[...]))
    fnew = fp / (nf + 1e-8) * nn
    fout[...] = fnew
    if qout is not None:
        qout[...] = jnp.dot(fnew, wqn[...], preferred_element_type=F32)


def _nodee(parts, f, wpz, wpf, ng, nb, expand, wqn):
    din = f.shape[1]
    full = lambda s: pl.BlockSpec(s, lambda i: tuple(0 for _ in s))
    in_specs = [
        pl.BlockSpec((2, BN, C), lambda i: (0, i, 0)),
        pl.BlockSpec((BN, din), lambda i: (i, 0)),
        full((CKV, C)), full((din, C)), full((1, C)), full((1, C)),
        full((H, CKV)),
    ]
    out_specs = [pl.BlockSpec((BN, C), lambda i: (i, 0))]
    out_shape = [jax.ShapeDtypeStruct((N, C), F32)]
    if wqn is not None:
        in_specs.append(full((C, CKV)))
        out_specs.append(pl.BlockSpec((BN, CKV), lambda i: (i, 0)))
        out_shape.append(jax.ShapeDtypeStruct((N, CKV), F32))
        body = _nodee_body
        args = (parts, f, wpz, wpf, ng, nb, expand, wqn)
    else:
        def body(parts, f, wpz, wpf, ng, nb, expand, fout):
            _nodee_body(parts, f, wpz, wpf, ng, nb, expand, None, fout, None)
        args = (parts, f, wpz, wpf, ng, nb, expand)
    return pl.pallas_call(
        body,
        grid=(N // BN,),
        in_specs=in_specs,
        out_specs=out_specs,
        out_shape=out_shape,
    )(*args)


def _outer_body(h, fs, selh, self_, g):
    g[...] = jnp.dot(h[...], selh[...], preferred_element_type=F32) * jnp.dot(
        fs[...], self_[...], preferred_element_type=F32)


def _outer(h, fs, selh, self_):
    nblk = E // BE
    eb = lambda d: pl.BlockSpec((BE, d), lambda i: (i, 0))
    full = lambda s: pl.BlockSpec(s, lambda i: (0, 0))
    return pl.pallas_call(
        _outer_body,
        grid=(nblk,),
        in_specs=[eb(MID), eb(C), full((MID, MID * C)), full((C, MID * C))],
        out_specs=eb(MID * C),
        out_shape=jax.ShapeDtypeStruct((E, MID * C), F32),
    )(h, fs, selh, self_)


def _final_body(parts, f, wf, wself, out, pooled):
    p = parts[...]
    a = p[0] + p[1]
    o = jnp.dot(a, wf[...], preferred_element_type=F32) + jnp.dot(
        f[...], wself[...], preferred_element_type=F32)
    out[...] = o
    bm = jnp.max(o, axis=0, keepdims=True)
    i = pl.program_id(0)

    @pl.when(i == 0)
    def _():
        pooled[...] = bm

    @pl.when(i > 0)
    def _():
        pooled[...] = jnp.maximum(pooled[...], bm)


def _final(parts, f, wf, wself):
    full = lambda s: pl.BlockSpec(s, lambda i: (0, 0))
    return pl.pallas_call(
        _final_body,
        grid=(N // BN,),
        in_specs=[
            pl.BlockSpec((2, BN, MID * C), lambda i: (0, i, 0)),
            pl.BlockSpec((BN, C), lambda i: (i, 0)),
            full((MID * C, OUT)), full((C, OUT)),
        ],
        out_specs=[pl.BlockSpec((BN, OUT), lambda i: (i, 0)), full((1, OUT))],
        out_shape=[
            jax.ShapeDtypeStruct((N, OUT), F32),
            jax.ShapeDtypeStruct((1, OUT), F32),
        ],
    )(parts, f, wf, wself)


def _sc_gather(table, idx):
    d = table.shape[1]
    e = idx.shape[0]
    nch = e // CH
    iters = (nch + NW - 1) // NW
    tiled = d % 128 == 0
    mesh = plsc.VectorSubcoreMesh(core_axis_name="c", subcore_axis_name="s")

    @functools.partial(
        pl.kernel,
        out_type=jax.ShapeDtypeStruct((e, d), F32),
        mesh=mesh,
        compiler_params=pltpu.CompilerParams(use_tc_tiling_on_sc=tiled),
        scratch_types=[
            pltpu.VMEM((CH,), jnp.int32),
            pltpu.VMEM((CH, d), F32),
            pltpu.SemaphoreType.DMA,
        ],
    )
    def k(table_hbm, idx_hbm, out_hbm, idx_v, rows_v, sem):
        wid = lax.axis_index("s") * 2 + lax.axis_index("c")

        def body(c, _):
            chunk = wid + NW * c

            @pl.when(chunk < nch)
            def _():
                base = chunk * CH
                pltpu.sync_copy(idx_hbm.at[pl.ds(base, CH)], idx_v)
                pltpu.async_copy(table_hbm.at[idx_v], rows_v, sem).wait()
                pltpu.sync_copy(rows_v, out_hbm.at[pl.ds(base, CH)])

            return 0

        lax.fori_loop(0, iters, body, 0)

    return k(table, idx)


def _sc_scatter_add(vals, dst, fc, fchunks):
    e = dst.shape[0]
    half = e // 2
    nch = half // CH
    iters = (nch + 15) // 16
    zc = 40
    nrc = N // zc
    riters = (nrc + 15) // 16
    ftot = vals.shape[1]
    tiled = fc % 128 == 0
    mesh = plsc.VectorSubcoreMesh(core_axis_name="c", subcore_axis_name="s")

    @functools.partial(
        pl.kernel,
        out_type=jax.ShapeDtypeStruct((2, N, ftot), F32),
        mesh=mesh,
        compiler_params=pltpu.CompilerParams(use_tc_tiling_on_sc=tiled),
        scratch_types=[
            pltpu.VMEM((2, CH), jnp.int32),
            pltpu.VMEM((2, CH, fc), F32),
            pltpu.VMEM((zc, fc), F32),
            pltpu.VMEM_SHARED((N, fc), F32),
            pltpu.SemaphoreType.DMA,
            pltpu.SemaphoreType.DMA,
        ],
    )
    def k(vals_hbm, dst_hbm, out_hbm, idx_v, v_v, zbuf, acc, sem_d, sem_v):
        cid = lax.axis_index("c")
        sid = lax.axis_index("s")
        for rr in range(zc):
            for j in range(fc // 16):
                zbuf[rr, pl.ds(j * 16, 16)] = jnp.zeros((16,), F32)
        for p in range(fchunks):
            fo = p * fc

            def zbody(cz, _):
                rc = sid + 16 * cz

                @pl.when(rc < nrc)
                def _():
                    pltpu.sync_copy(zbuf, acc.at[pl.ds(rc * zc, zc)])

                return 0

            lax.fori_loop(0, riters, zbody, 0)
            plsc.subcore_barrier()

            def dslc(c):
                chunk = sid + 16 * c
                base = cid * half + chunk * CH
                return (dst_hbm.at[pl.ds(base, CH)],
                        vals_hbm.at[pl.ds(base, CH), pl.ds(fo, fc)], chunk)

            d0, v0, ch0 = dslc(0)

            @pl.when(ch0 < nch)
            def _():
                pltpu.async_copy(d0, idx_v.at[0], sem_d)
                pltpu.async_copy(v0, v_v.at[0], sem_v)

            def body2(j2, _):
                for b in range(2):
                    c = 2 * j2 + b
                    dsl, vsl, chk = dslc(c)
                    pred = chk < nch

                    @pl.when(pred)
                    def _():
                        pltpu.make_async_copy(dsl, idx_v.at[b], sem_d).wait()
                        pltpu.make_async_copy(vsl, v_v.at[b], sem_v).wait()

                    dsn, vsn, chn = dslc(c + 1)

                    @pl.when(chn < nch)
                    def _():
                        pltpu.async_copy(dsn, idx_v.at[1 - b], sem_d)
                        pltpu.async_copy(vsn, v_v.at[1 - b], sem_v)

                    @pl.when(pred)
                    def _():
                        pltpu.sync_copy(v_v.at[b], acc.at[idx_v.at[b]],
                                        add=True)
                return 0

            lax.fori_loop(0, iters // 2, body2, 0)
            plsc.subcore_barrier()

            def wbody(cw, _):
                rc = sid + 16 * cw

                @pl.when(rc < nrc)
                def _():
                    pltpu.sync_copy(
                        acc.at[pl.ds(rc * zc, zc)],
                        out_hbm.at[cid, pl.ds(rc * zc, zc), pl.ds(fo, fc)])

                return 0

            lax.fori_loop(0, riters, wbody, 0)
            plsc.subcore_barrier()

    return k(vals, dst)


def _layer(f, q, src, dst, h, w3, wproj, ng, nb, wq_next, consts):
    sel1, summ, pair, expand = consts
    din = f.shape[1]
    fs = _sc_gather(f, src)
    qd = _sc_gather(q, dst)
    w3all = w3.transpose(2, 0, 1).reshape(din, MID * 2 * CKV)
    lg, v, gmax = _edgec(h, fs, qd, w3all, sel1, summ, pair)
    scv = _edged(lg, v, gmax, expand)
    parts = _sc_scatter_add(scv, dst, C, 1)
    wpz = wproj[:CKV]
    wpf = wproj[CKV:]
    return _nodee(parts, f, wpz, wpf, ng.reshape(1, C), nb.reshape(1, C),
                  expand, wq_next)


def kernel(node_feats, edge_index, edge_w, rel_pos,
           l0_rw1, l0_rb1, l0_rg1, l0_rbt1, l0_rw2, l0_rb2, l0_rg2, l0_rbt2,
           l0_w3, l0_wq, l0_wproj, l0_ng, l0_nb,
           l1_rw1, l1_rb1, l1_rg1, l1_rbt1, l1_rw2, l1_rb2, l1_rg2, l1_rbt2,
           l1_w3, l1_wq, l1_wproj, l1_ng, l1_nb,
           f_rw1, f_rb1, f_rg1, f_rbt1, f_rw2, f_rb2, f_rg2, f_rbt2,
           f_w3, f_wself):
    src = edge_index[0]
    dst = edge_index[1]

    w1 = jnp.stack([l0_rw1, l1_rw1, f_rw1])
    b1 = jnp.stack([l0_rb1, l1_rb1, f_rb1])
    g1 = jnp.stack([l0_rg1, l1_rg1, f_rg1])
    bt1 = jnp.stack([l0_rbt1, l1_rbt1, f_rbt1])
    w2 = jnp.stack([l0_rw2, l1_rw2, f_rw2])
    b2 = jnp.stack([l0_rb2, l1_rb2, f_rb2])
    g2 = jnp.stack([l0_rg2, l1_rg2, f_rg2])
    bt2 = jnp.stack([l0_rbt2, l1_rbt2, f_rbt2])
    h0, h1, hf = _radial(edge_w, rel_pos, w1, b1, g1, bt1, w2, b2, g2, bt2)

    sel1 = jnp.kron(jnp.eye(MID, dtype=F32), jnp.ones((1, 2 * CKV), F32))
    summ = jnp.tile(jnp.eye(2 * CKV, dtype=F32), (MID, 1))
    pair = jnp.kron(jnp.eye(H, dtype=F32), jnp.ones((DH, 1), F32))
    expand = jnp.kron(jnp.eye(H, dtype=F32), jnp.ones((1, DH), F32))
    consts = (sel1, summ, pair, expand)

    q0 = _mm(node_feats, l0_wq)
    f1, q1 = _layer(node_feats, q0, src, dst, h0, l0_w3, l0_wproj,
                    l0_ng, l0_nb, l1_wq, consts)
    f2 = _layer(f1, q1, src, dst, h1, l1_w3, l1_wproj,
                l1_ng, l1_nb, None, consts)[0]

    fsf = _sc_gather(f2, src)
    selh = jnp.kron(jnp.eye(MID, dtype=F32), jnp.ones((1, C), F32))
    self_ = jnp.tile(jnp.eye(C, dtype=F32), (1, MID))
    g = _outer(hf, fsf, selh, self_)
    partsf = _sc_scatter_add(g, dst, 128, 4)
    wf = f_w3.transpose(0, 2, 1).reshape(MID * C, OUT)
    out, pooled = _final(partsf, f2, wf, f_wself)
    return (out[:, :, None], pooled)


# speedup vs baseline: 11.6092x; 1.0134x over previous
import functools
import jax
import jax.numpy as jnp
from jax import lax
from jax.experimental import pallas as pl
from jax.experimental.pallas import tpu as pltpu
from jax.experimental.pallas import tpu_sc as plsc

F32 = jnp.float32

N = 10000
E = 160000
DIN = 128
MID = 16
C = 32
CKV = 16
H = 8
DH = 2
OUT = 1280

BE = 800
BN = 400
NW = 32
CH = 128


def _ln(x, g, b):
    mu = jnp.mean(x, axis=-1, keepdims=True)
    var = jnp.mean((x - mu) * (x - mu), axis=-1, keepdims=True)
    return (x - mu) * lax.rsqrt(var + 1e-5) * g + b


def _radial_body(ew, rp, w1, b1, g1, bt1, w2, b2, g2, bt2, h0, h1, hf):
    rpv = rp[...]
    r = jnp.sqrt(jnp.sum(rpv * rpv, axis=1, keepdims=True))
    es = jnp.concatenate([ew[...], r], axis=1)
    outs = (h0, h1, hf)
    for j in range(3):
        x = jnp.dot(es, w1[j], preferred_element_type=F32) + b1[j : j + 1]
        x = jax.nn.relu(_ln(x, g1[j : j + 1], bt1[j : j + 1]))
        x = jnp.dot(x, w2[j], preferred_element_type=F32) + b2[j : j + 1]
        x = jax.nn.relu(_ln(x, g2[j : j + 1], bt2[j : j + 1]))
        outs[j][...] = x


def _radial(ew, rp, w1, b1, g1, bt1, w2, b2, g2, bt2):
    nblk = E // BE
    eb = lambda d: pl.BlockSpec((BE, d), lambda i: (i, 0))
    full = lambda s: pl.BlockSpec(s, lambda i: tuple(0 for _ in s))
    return pl.pallas_call(
        _radial_body,
        grid=(nblk,),
        in_specs=[
            eb(4), eb(3),
            full((3, 5, MID)), full((3, MID)), full((3, MID)), full((3, MID)),
            full((3, MID, MID)), full((3, MID)), full((3, MID)), full((3, MID)),
        ],
        out_specs=[eb(MID), eb(MID), eb(MID)],
        out_shape=[jax.ShapeDtypeStruct((E, MID), F32)] * 3,
    )(ew, rp, w1, b1, g1, bt1, w2, b2, g2, bt2)


def _mm_body(x, w, o):
    o[...] = jnp.dot(x[...], w[...], preferred_element_type=F32)


def _mm(x, w):
    n, k = x.shape
    m = w.shape[1]
    return pl.pallas_call(
        _mm_body,
        grid=(n // BN,),
        in_specs=[
            pl.BlockSpec((BN, k), lambda i: (i, 0)),
            pl.BlockSpec((k, m), lambda i: (0, 0)),
        ],
        out_specs=pl.BlockSpec((BN, m), lambda i: (i, 0)),
        out_shape=jax.ShapeDtypeStruct((n, m), F32),
    )(x, w)


def _edgec_body(h, fs, qd, w3all, sel1, summ, pair, lg, v, gmax):
    t = jnp.dot(fs[...], w3all[...], preferred_element_type=F32)
    kvw = t * jnp.dot(h[...], sel1[...], preferred_element_type=F32)
    kv = jnp.dot(kvw, summ[...], preferred_element_type=F32)
    k = kv[:, :CKV]
    vv = kv[:, CKV:]
    prod = k * qd[...]
    lgb = jnp.dot(prod, pair[...], preferred_element_type=F32) * (DH ** -0.5)
    lg[...] = lgb
    v[...] = vv
    bm = jnp.max(lgb, axis=0, keepdims=True)
    i = pl.program_id(0)

    @pl.when(i == 0)
    def _():
        gmax[...] = bm

    @pl.when(i > 0)
    def _():
        gmax[...] = jnp.maximum(gmax[...], bm)


def _edgec(h, fs, qd, w3all, sel1, summ, pair):
    din = fs.shape[1]
    nblk = E // BE
    eb = lambda d: pl.BlockSpec((BE, d), lambda i: (i, 0))
    full = lambda s: pl.BlockSpec(s, lambda i: (0, 0))
    return pl.pallas_call(
        _edgec_body,
        grid=(nblk,),
        in_specs=[
            eb(MID), eb(din), eb(CKV),
            full((din, 2 * CKV * MID)), full((MID, 2 * CKV * MID)),
            full((2 * CKV * MID, 2 * CKV)), full((CKV, H)),
        ],
        out_specs=[eb(H), eb(CKV), full((1, H))],
        out_shape=[
            jax.ShapeDtypeStruct((E, H), F32),
            jax.ShapeDtypeStruct((E, CKV), F32),
            jax.ShapeDtypeStruct((1, H), F32),
        ],
    )(h, fs, qd, w3all, sel1, summ, pair)


def _edged_body(lg, v, gmax, expand, scv):
    ex = jnp.exp(lg[...] - gmax[...])
    num = jnp.dot(ex, expand[...], preferred_element_type=F32) * v[...]
    z8 = jnp.zeros((BE, H), F32)
    scv[...] = jnp.concatenate([ex, num, z8], axis=1)


def _edged(lg, v, gmax, expand):
    nblk = E // BE
    eb = lambda d: pl.BlockSpec((BE, d), lambda i: (i, 0))
    full = lambda s: pl.BlockSpec(s, lambda i: (0, 0))
    return pl.pallas_call(
        _edged_body,
        grid=(nblk,),
        in_specs=[eb(H), eb(CKV), full((1, H)), full((H, CKV))],
        out_specs=eb(C),
        out_shape=jax.ShapeDtypeStruct((E, C), F32),
    )(lg, v, gmax, expand)


def _nodee_body(parts, f, wpz, wpf, ng, nb, expand, wqn, fout, qout):
    p = parts[...]
    s = p[0] + p[1]
    den = s[:, :H]
    num = s[:, H : H + CKV]
    dexp = jnp.dot(den, expand[...], preferred_element_type=F32) + 1e-30
    z = num / dexp
    fp = jnp.dot(z, wpz[...], preferred_element_type=F32) + jnp.dot(
        f[...], wpf[...], preferred_element_type=F32)
    nf = jnp.abs(fp)
    nn = jax.nn.relu(_ln(nf, ng[...], nb[...]))
    fnew = fp / (nf + 1e-8) * nn
    fout[...] = fnew
    if qout is not None:
        qout[...] = jnp.dot(fnew, wqn[...], preferred_element_type=F32)


def _nodee(parts, f, wpz, wpf, ng, nb, expand, wqn):
    din = f.shape[1]
    full = lambda s: pl.BlockSpec(s, lambda i: tuple(0 for _ in s))
    in_specs = [
        pl.BlockSpec((2, BN, C), lambda i: (0, i, 0)),
        pl.BlockSpec((BN, din), lambda i: (i, 0)),
        full((CKV, C)), full((din, C)), full((1, C)), full((1, C)),
        full((H, CKV)),
    ]
    out_specs = [pl.BlockSpec((BN, C), lambda i: (i, 0))]
    out_shape = [jax.ShapeDtypeStruct((N, C), F32)]
    if wqn is not None:
        in_specs.append(full((C, CKV)))
        out_specs.append(pl.BlockSpec((BN, CKV), lambda i: (i, 0)))
        out_shape.append(jax.ShapeDtypeStruct((N, CKV), F32))
        body = _nodee_body
        args = (parts, f, wpz, wpf, ng, nb, expand, wqn)
    else:
        def body(parts, f, wpz, wpf, ng, nb, expand, fout):
            _nodee_body(parts, f, wpz, wpf, ng, nb, expand, None, fout, None)
        args = (parts, f, wpz, wpf, ng, nb, expand)
    return pl.pallas_call(
        body,
        grid=(N // BN,),
        in_specs=in_specs,
        out_specs=out_specs,
        out_shape=out_shape,
    )(*args)


def _outer_body(h, fs, selh, self_, g):
    g[...] = jnp.dot(h[...], selh[...], preferred_element_type=F32) * jnp.dot(
        fs[...], self_[...], preferred_element_type=F32)


def _outer(h, fs, selh, self_):
    nblk = E // BE
    eb = lambda d: pl.BlockSpec((BE, d), lambda i: (i, 0))
    full = lambda s: pl.BlockSpec(s, lambda i: (0, 0))
    return pl.pallas_call(
        _outer_body,
        grid=(nblk,),
        in_specs=[eb(MID), eb(C), full((MID, MID * C)), full((C, MID * C))],
        out_specs=eb(MID * C),
        out_shape=jax.ShapeDtypeStruct((E, MID * C), F32),
    )(h, fs, selh, self_)


def _final_body(parts, f, wf, wself, out, pooled):
    p = parts[...]
    a = p[0] + p[1]
    o = jnp.dot(a, wf[...], preferred_element_type=F32) + jnp.dot(
        f[...], wself[...], preferred_element_type=F32)
    out[...] = o
    bm = jnp.max(o, axis=0, keepdims=True)
    i = pl.program_id(0)

    @pl.when(i == 0)
    def _():
        pooled[...] = bm

    @pl.when(i > 0)
    def _():
        pooled[...] = jnp.maximum(pooled[...], bm)


def _final(parts, f, wf, wself):
    full = lambda s: pl.BlockSpec(s, lambda i: (0, 0))
    return pl.pallas_call(
        _final_body,
        grid=(N // BN,),
        in_specs=[
            pl.BlockSpec((2, BN, MID * C), lambda i: (0, i, 0)),
            pl.BlockSpec((BN, C), lambda i: (i, 0)),
            full((MID * C, OUT)), full((C, OUT)),
        ],
        out_specs=[pl.BlockSpec((BN, OUT), lambda i: (i, 0)), full((1, OUT))],
        out_shape=[
            jax.ShapeDtypeStruct((N, OUT), F32),
            jax.ShapeDtypeStruct((1, OUT), F32),
        ],
    )(parts, f, wf, wself)


def _sc_gather(table, idx):
    d = table.shape[1]
    e = idx.shape[0]
    nch = e // CH
    iters = (nch + NW - 1) // NW
    tiled = d % 128 == 0
    mesh = plsc.VectorSubcoreMesh(core_axis_name="c", subcore_axis_name="s")

    @functools.partial(
        pl.kernel,
        out_type=jax.ShapeDtypeStruct((e, d), F32),
        mesh=mesh,
        compiler_params=pltpu.CompilerParams(use_tc_tiling_on_sc=tiled),
        scratch_types=[
            pltpu.VMEM((2, CH), jnp.int32),
            pltpu.VMEM((2, CH, d), F32),
            pltpu.SemaphoreType.DMA,
            pltpu.SemaphoreType.DMA,
            pltpu.SemaphoreType.DMA,
            pltpu.SemaphoreType.DMA,
        ],
    )
    def k(table_hbm, idx_hbm, out_hbm, idx_v, rows_v, sem_i, sem_g, so0, so1):
        wid = lax.axis_index("s") * 2 + lax.axis_index("c")
        sem_o = (so0, so1)

        def islc(c):
            base = jnp.minimum(wid + NW * c, nch - 1) * CH
            return idx_hbm.at[pl.ds(base, CH)], out_hbm.at[pl.ds(base, CH)]

        i0, _o0 = islc(0)
        pltpu.async_copy(i0, idx_v.at[0], sem_i)

        def body2(j2, _):
            for b in range(2):
                c = 2 * j2 + b
                isl, osl = islc(c)
                pltpu.make_async_copy(isl, idx_v.at[b], sem_i).wait()
                isn, _osn = islc(c + 1)

                @pl.when(c + 1 < iters)
                def _():
                    pltpu.async_copy(isn, idx_v.at[1 - b], sem_i)

                @pl.when(c >= 2)
                def _():
                    _ipre, opre = islc(c - 2)
                    pltpu.make_async_copy(rows_v.at[b], opre, sem_o[b]).wait()

                pltpu.async_copy(
                    table_hbm.at[idx_v.at[b]], rows_v.at[b], sem_g).wait()
                pltpu.async_copy(rows_v.at[b], osl, sem_o[b])
            return 0

        lax.fori_loop(0, iters // 2, body2, 0)
        for c in (iters - 2, iters - 1):
            b = c % 2
            _il, ol = islc(c)
            pltpu.make_async_copy(rows_v.at[b], ol, sem_o[b]).wait()

    return k(table, idx)


def _sc_scatter_add(vals, dst, fc, fchunks):
    e = dst.shape[0]
    half = e // 2
    nch = half // CH
    iters = (nch + 15) // 16
    zc = 40
    nrc = N // zc
    riters = (nrc + 15) // 16
    ftot = vals.shape[1]
    tiled = fc % 128 == 0
    mesh = plsc.VectorSubcoreMesh(core_axis_name="c", subcore_axis_name="s")

    @functools.partial(
        pl.kernel,
        out_type=jax.ShapeDtypeStruct((2, N, ftot), F32),
        mesh=mesh,
        compiler_params=pltpu.CompilerParams(use_tc_tiling_on_sc=tiled),
        scratch_types=[
            pltpu.VMEM((2, CH), jnp.int32),
            pltpu.VMEM((2, CH, fc), F32),
            pltpu.VMEM((zc, fc), F32),
            pltpu.VMEM_SHARED((N, fc), F32),
            pltpu.SemaphoreType.DMA,
            pltpu.SemaphoreType.DMA,
        ],
    )
    def k(vals_hbm, dst_hbm, out_hbm, idx_v, v_v, zbuf, acc, sem_d, sem_v):
        cid = lax.axis_index("c")
        sid = lax.axis_index("s")
        for rr in range(zc):
            for j in range(fc // 16):
                zbuf[rr, pl.ds(j * 16, 16)] = jnp.zeros((16,), F32)
        for p in range(fchunks):
            fo = p * fc

            def zbody(cz, _):
                rc = sid + 16 * cz

                @pl.when(rc < nrc)
                def _():
                    pltpu.sync_copy(zbuf, acc.at[pl.ds(rc * zc, zc)])

                return 0

            lax.fori_loop(0, riters, zbody, 0)
            plsc.subcore_barrier()

            def dslc(c):
                chunk = sid + 16 * c
                base = cid * half + chunk * CH
                return (dst_hbm.at[pl.ds(base, CH)],
                        vals_hbm.at[pl.ds(base, CH), pl.ds(fo, fc)], chunk)

            d0, v0, ch0 = dslc(0)

            @pl.when(ch0 < nch)
            def _():
                pltpu.async_copy(d0, idx_v.at[0], sem_d)
                pltpu.async_copy(v0, v_v.at[0], sem_v)

            def body2(j2, _):
                for b in range(2):
                    c = 2 * j2 + b
                    dsl, vsl, chk = dslc(c)
                    pred = chk < nch

                    @pl.when(pred)
                    def _():
                        pltpu.make_async_copy(dsl, idx_v.at[b], sem_d).wait()
                        pltpu.make_async_copy(vsl, v_v.at[b], sem_v).wait()

                    dsn, vsn, chn = dslc(c + 1)

                    @pl.when(chn < nch)
                    def _():
                        pltpu.async_copy(dsn, idx_v.at[1 - b], sem_d)
                        pltpu.async_copy(vsn, v_v.at[1 - b], sem_v)

                    @pl.when(pred)
                    def _():
                        pltpu.sync_copy(v_v.at[b], acc.at[idx_v.at[b]],
                                        add=True)
                return 0

            lax.fori_loop(0, iters // 2, body2, 0)
            plsc.subcore_barrier()

            def wbody(cw, _):
                rc = sid + 16 * cw

                @pl.when(rc < nrc)
                def _():
                    pltpu.sync_copy(
                        acc.at[pl.ds(rc * zc, zc)],
                        out_hbm.at[cid, pl.ds(rc * zc, zc), pl.ds(fo, fc)])

                return 0

            lax.fori_loop(0, riters, wbody, 0)
            plsc.subcore_barrier()

    return k(vals, dst)


def _layer(f, q, src, dst, h, w3, wproj, ng, nb, wq_next, consts):
    sel1, summ, pair, expand = consts
    din = f.shape[1]
    fs = _sc_gather(f, src)
    qd = _sc_gather(q, dst)
    w3all = w3.transpose(2, 0, 1).reshape(din, MID * 2 * CKV)
    lg, v, gmax = _edgec(h, fs, qd, w3all, sel1, summ, pair)
    scv = _edged(lg, v, gmax, expand)
    parts = _sc_scatter_add(scv, dst, C, 1)
    wpz = wproj[:CKV]
    wpf = wproj[CKV:]
    return _nodee(parts, f, wpz, wpf, ng.reshape(1, C), nb.reshape(1, C),
                  expand, wq_next)


def kernel(node_feats, edge_index, edge_w, rel_pos,
           l0_rw1, l0_rb1, l0_rg1, l0_rbt1, l0_rw2, l0_rb2, l0_rg2, l0_rbt2,
           l0_w3, l0_wq, l0_wproj, l0_ng, l0_nb,
           l1_rw1, l1_rb1, l1_rg1, l1_rbt1, l1_rw2, l1_rb2, l1_rg2, l1_rbt2,
           l1_w3, l1_wq, l1_wproj, l1_ng, l1_nb,
           f_rw1, f_rb1, f_rg1, f_rbt1, f_rw2, f_rb2, f_rg2, f_rbt2,
           f_w3, f_wself):
    src = edge_index[0]
    dst = edge_index[1]

    w1 = jnp.stack([l0_rw1, l1_rw1, f_rw1])
    b1 = jnp.stack([l0_rb1, l1_rb1, f_rb1])
    g1 = jnp.stack([l0_rg1, l1_rg1, f_rg1])
    bt1 = jnp.stack([l0_rbt1, l1_rbt1, f_rbt1])
    w2 = jnp.stack([l0_rw2, l1_rw2, f_rw2])
    b2 = jnp.stack([l0_rb2, l1_rb2, f_rb2])
    g2 = jnp.stack([l0_rg2, l1_rg2, f_rg2])
    bt2 = jnp.stack([l0_rbt2, l1_rbt2, f_rbt2])
    h0, h1, hf = _radial(edge_w, rel_pos, w1, b1, g1, bt1, w2, b2, g2, bt2)

    sel1 = jnp.kron(jnp.eye(MID, dtype=F32), jnp.ones((1, 2 * CKV), F32))
    summ = jnp.tile(jnp.eye(2 * CKV, dtype=F32), (MID, 1))
    pair = jnp.kron(jnp.eye(H, dtype=F32), jnp.ones((DH, 1), F32))
    expand = jnp.kron(jnp.eye(H, dtype=F32), jnp.ones((1, DH), F32))
    consts = (sel1, summ, pair, expand)

    q0 = _mm(node_feats, l0_wq)
    f1, q1 = _layer(node_feats, q0, src, dst, h0, l0_w3, l0_wproj,
                    l0_ng, l0_nb, l1_wq, consts)
    f2 = _layer(f1, q1, src, dst, h1, l1_w3, l1_wproj,
                l1_ng, l1_nb, None, consts)[0]

    fsf = _sc_gather(f2, src)
    selh = jnp.kron(jnp.eye(MID, dtype=F32), jnp.ones((1, C), F32))
    self_ = jnp.tile(jnp.eye(C, dtype=F32), (1, MID))
    g = _outer(hf, fsf, selh, self_)
    partsf = _sc_scatter_add(g, dst, 128, 4)
    wf = f_w3.transpose(0, 2, 1).reshape(MID * C, OUT)
    out, pooled = _final(partsf, f2, wf, f_wself)
    return (out[:, :, None], pooled)


# speedup vs baseline: 13.4783x; 1.1610x over previous
import functools
import jax
import jax.numpy as jnp
from jax import lax
from jax.experimental import pallas as pl
from jax.experimental.pallas import tpu as pltpu
from jax.experimental.pallas import tpu_sc as plsc

F32 = jnp.float32

N = 10000
E = 160000
DIN = 128
MID = 16
C = 32
CKV = 16
H = 8
DH = 2
OUT = 1280

BE = 1600
BN = 400
NW = 32
CH = 128


def _ln(x, g, b):
    mu = jnp.mean(x, axis=-1, keepdims=True)
    var = jnp.mean((x - mu) * (x - mu), axis=-1, keepdims=True)
    return (x - mu) * lax.rsqrt(var + 1e-5) * g + b


def _radial_body(ew, rp, w1, b1, g1, bt1, w2, b2, g2, bt2, h0, h1, hf):
    rpv = rp[...]
    r = jnp.sqrt(jnp.sum(rpv * rpv, axis=1, keepdims=True))
    es = jnp.concatenate([ew[...], r], axis=1)
    outs = (h0, h1, hf)
    for j in range(3):
        x = jnp.dot(es, w1[j], preferred_element_type=F32) + b1[j : j + 1]
        x = jax.nn.relu(_ln(x, g1[j : j + 1], bt1[j : j + 1]))
        x = jnp.dot(x, w2[j], preferred_element_type=F32) + b2[j : j + 1]
        x = jax.nn.relu(_ln(x, g2[j : j + 1], bt2[j : j + 1]))
        outs[j][...] = x


def _radial(ew, rp, w1, b1, g1, bt1, w2, b2, g2, bt2):
    nblk = E // BE
    eb = lambda d: pl.BlockSpec((BE, d), lambda i: (i, 0))
    full = lambda s: pl.BlockSpec(s, lambda i: tuple(0 for _ in s))
    return pl.pallas_call(
        _radial_body,
        grid=(nblk,),
        in_specs=[
            eb(4), eb(3),
            full((3, 5, MID)), full((3, MID)), full((3, MID)), full((3, MID)),
            full((3, MID, MID)), full((3, MID)), full((3, MID)), full((3, MID)),
        ],
        out_specs=[eb(MID), eb(MID), eb(MID)],
        out_shape=[jax.ShapeDtypeStruct((E, MID), F32)] * 3,
    )(ew, rp, w1, b1, g1, bt1, w2, b2, g2, bt2)


def _mm_body(x, w, o):
    o[...] = jnp.dot(x[...], w[...], preferred_element_type=F32)


def _mm(x, w):
    n, k = x.shape
    m = w.shape[1]
    return pl.pallas_call(
        _mm_body,
        grid=(n // BN,),
        in_specs=[
            pl.BlockSpec((BN, k), lambda i: (i, 0)),
            pl.BlockSpec((k, m), lambda i: (0, 0)),
        ],
        out_specs=pl.BlockSpec((BN, m), lambda i: (i, 0)),
        out_shape=jax.ShapeDtypeStruct((n, m), F32),
    )(x, w)


def _edgec_body(h, fs, qd, w3all, sel1, summ, pair, lg, v, gmax):
    t = jnp.dot(fs[...], w3all[...], preferred_element_type=F32)
    kvw = t * jnp.dot(h[...], sel1[...], preferred_element_type=F32)
    kv = jnp.dot(kvw, summ[...], preferred_element_type=F32)
    k = kv[:, :CKV]
    vv = kv[:, CKV:]
    prod = k * qd[...]
    lgb = jnp.dot(prod, pair[...], preferred_element_type=F32) * (DH ** -0.5)
    lg[...] = lgb
    v[...] = vv
    bm = jnp.max(lgb, axis=0, keepdims=True)
    i = pl.program_id(0)

    @pl.when(i == 0)
    def _():
        gmax[...] = bm

    @pl.when(i > 0)
    def _():
        gmax[...] = jnp.maximum(gmax[...], bm)


def _edgec(h, fs, qd, w3all, sel1, summ, pair):
    din = fs.shape[1]
    nblk = E // BE
    eb = lambda d: pl.BlockSpec((BE, d), lambda i: (i, 0))
    full = lambda s: pl.BlockSpec(s, lambda i: (0, 0))
    return pl.pallas_call(
        _edgec_body,
        grid=(nblk,),
        in_specs=[
            eb(MID), eb(din), eb(CKV),
            full((din, 2 * CKV * MID)), full((MID, 2 * CKV * MID)),
            full((2 * CKV * MID, 2 * CKV)), full((CKV, H)),
        ],
        out_specs=[eb(H), eb(CKV), full((1, H))],
        out_shape=[
            jax.ShapeDtypeStruct((E, H), F32),
            jax.ShapeDtypeStruct((E, CKV), F32),
            jax.ShapeDtypeStruct((1, H), F32),
        ],
    )(h, fs, qd, w3all, sel1, summ, pair)


def _edged_body(lg, v, gmax, expand, scv):
    ex = jnp.exp(lg[...] - gmax[...])
    num = jnp.dot(ex, expand[...], preferred_element_type=F32) * v[...]
    z8 = jnp.zeros((BE, H), F32)
    scv[...] = jnp.concatenate([ex, num, z8], axis=1)


def _edged(lg, v, gmax, expand):
    nblk = E // BE
    eb = lambda d: pl.BlockSpec((BE, d), lambda i: (i, 0))
    full = lambda s: pl.BlockSpec(s, lambda i: (0, 0))
    return pl.pallas_call(
        _edged_body,
        grid=(nblk,),
        in_specs=[eb(H), eb(CKV), full((1, H)), full((H, CKV))],
        out_specs=eb(C),
        out_shape=jax.ShapeDtypeStruct((E, C), F32),
    )(lg, v, gmax, expand)


def _nodee_body(parts, f, wpz, wpf, ng, nb, expand, wqn, fout, qout):
    p = parts[...]
    s = p[0] + p[1]
    den = s[:, :H]
    num = s[:, H : H + CKV]
    dexp = jnp.dot(den, expand[...], preferred_element_type=F32) + 1e-30
    z = num / dexp
    fp = jnp.dot(z, wpz[...], preferred_element_type=F32) + jnp.dot(
        f[...], wpf[...], preferred_element_type=F32)
    nf = jnp.abs(fp)
    nn = jax.nn.relu(_ln(nf, ng[...], nb[...]))
    fnew = fp / (nf + 1e-8) * nn
    fout[...] = fnew
    if qout is not None:
        qout[...] = jnp.dot(fnew, wqn[...], preferred_element_type=F32)


def _nodee(parts, f, wpz, wpf, ng, nb, expand, wqn):
    din = f.shape[1]
    full = lambda s: pl.BlockSpec(s, lambda i: tuple(0 for _ in s))
    in_specs = [
        pl.BlockSpec((2, BN, C), lambda i: (0, i, 0)),
        pl.BlockSpec((BN, din), lambda i: (i, 0)),
        full((CKV, C)), full((din, C)), full((1, C)), full((1, C)),
        full((H, CKV)),
    ]
    out_specs = [pl.BlockSpec((BN, C), lambda i: (i, 0))]
    out_shape = [jax.ShapeDtypeStruct((N, C), F32)]
    if wqn is not None:
        in_specs.append(full((C, CKV)))
        out_specs.append(pl.BlockSpec((BN, CKV), lambda i: (i, 0)))
        out_shape.append(jax.ShapeDtypeStruct((N, CKV), F32))
        body = _nodee_body
        args = (parts, f, wpz, wpf, ng, nb, expand, wqn)
    else:
        def body(parts, f, wpz, wpf, ng, nb, expand, fout):
            _nodee_body(parts, f, wpz, wpf, ng, nb, expand, None, fout, None)
        args = (parts, f, wpz, wpf, ng, nb, expand)
    return pl.pallas_call(
        body,
        grid=(N // BN,),
        in_specs=in_specs,
        out_specs=out_specs,
        out_shape=out_shape,
    )(*args)


def _outer_body(h, fs, selh, self_, g):
    g[...] = jnp.dot(h[...], selh[...], preferred_element_type=F32) * jnp.dot(
        fs[...], self_[...], preferred_element_type=F32)


def _outer(h, fs, selh, self_):
    nblk = E // BE
    eb = lambda d: pl.BlockSpec((BE, d), lambda i: (i, 0))
    full = lambda s: pl.BlockSpec(s, lambda i: (0, 0))
    return pl.pallas_call(
        _outer_body,
        grid=(nblk,),
        in_specs=[eb(MID), eb(C), full((MID, MID * C)), full((C, MID * C))],
        out_specs=eb(MID * C),
        out_shape=jax.ShapeDtypeStruct((E, MID * C), F32),
    )(h, fs, selh, self_)


def _final_body(parts, f, wf, wself, out, pooled):
    p = parts[...]
    a = p[0] + p[1]
    o = jnp.dot(a, wf[...], preferred_element_type=F32) + jnp.dot(
        f[...], wself[...], preferred_element_type=F32)
    out[...] = o
    bm = jnp.max(o, axis=0, keepdims=True)
    i = pl.program_id(0)

    @pl.when(i == 0)
    def _():
        pooled[...] = bm

    @pl.when(i > 0)
    def _():
        pooled[...] = jnp.maximum(pooled[...], bm)


def _final(parts, f, wf, wself):
    full = lambda s: pl.BlockSpec(s, lambda i: (0, 0))
    return pl.pallas_call(
        _final_body,
        grid=(N // BN,),
        in_specs=[
            pl.BlockSpec((2, BN, MID * C), lambda i: (0, i, 0)),
            pl.BlockSpec((BN, C), lambda i: (i, 0)),
            full((MID * C, OUT)), full((C, OUT)),
        ],
        out_specs=[pl.BlockSpec((BN, OUT), lambda i: (i, 0)), full((1, OUT))],
        out_shape=[
            jax.ShapeDtypeStruct((N, OUT), F32),
            jax.ShapeDtypeStruct((1, OUT), F32),
        ],
    )(parts, f, wf, wself)


def _sc_gather(table, idx):
    d = table.shape[1]
    e = idx.shape[0]
    nch = e // CH
    iters = (nch + NW - 1) // NW
    tiled = d % 128 == 0
    mesh = plsc.VectorSubcoreMesh(core_axis_name="c", subcore_axis_name="s")

    @functools.partial(
        pl.kernel,
        out_type=jax.ShapeDtypeStruct((e, d), F32),
        mesh=mesh,
        compiler_params=pltpu.CompilerParams(use_tc_tiling_on_sc=tiled),
        scratch_types=[
            pltpu.VMEM((2, CH), jnp.int32),
            pltpu.VMEM((2, CH, d), F32),
            pltpu.SemaphoreType.DMA,
            pltpu.SemaphoreType.DMA,
            pltpu.SemaphoreType.DMA,
            pltpu.SemaphoreType.DMA,
        ],
    )
    def k(table_hbm, idx_hbm, out_hbm, idx_v, rows_v, sem_i, sem_g, so0, so1):
        wid = lax.axis_index("s") * 2 + lax.axis_index("c")
        sem_o = (so0, so1)

        def islc(c):
            base = jnp.minimum(wid + NW * c, nch - 1) * CH
            return idx_hbm.at[pl.ds(base, CH)], out_hbm.at[pl.ds(base, CH)]

        i0, _o0 = islc(0)
        pltpu.async_copy(i0, idx_v.at[0], sem_i)

        def body2(j2, _):
            for b in range(2):
                c = 2 * j2 + b
                isl, osl = islc(c)
                pltpu.make_async_copy(isl, idx_v.at[b], sem_i).wait()
                isn, _osn = islc(c + 1)

                @pl.when(c + 1 < iters)
                def _():
                    pltpu.async_copy(isn, idx_v.at[1 - b], sem_i)

                @pl.when(c >= 2)
                def _():
                    _ipre, opre = islc(c - 2)
                    pltpu.make_async_copy(rows_v.at[b], opre, sem_o[b]).wait()

                pltpu.async_copy(
                    table_hbm.at[idx_v.at[b]], rows_v.at[b], sem_g).wait()
                pltpu.async_copy(rows_v.at[b], osl, sem_o[b])
            return 0

        lax.fori_loop(0, iters // 2, body2, 0)
        for c in (iters - 2, iters - 1):
            b = c % 2
            _il, ol = islc(c)
            pltpu.make_async_copy(rows_v.at[b], ol, sem_o[b]).wait()

    return k(table, idx)


def _sc_scatter_add(vals, dst, fc, fchunks):
    e = dst.shape[0]
    half = e // 2
    nch = half // CH
    iters = (nch + 15) // 16
    zc = 40
    nrc = N // zc
    riters = (nrc + 15) // 16
    ftot = vals.shape[1]
    tiled = fc % 128 == 0
    mesh = plsc.VectorSubcoreMesh(core_axis_name="c", subcore_axis_name="s")

    @functools.partial(
        pl.kernel,
        out_type=jax.ShapeDtypeStruct((2, N, ftot), F32),
        mesh=mesh,
        compiler_params=pltpu.CompilerParams(use_tc_tiling_on_sc=tiled),
        scratch_types=[
            pltpu.VMEM((2, CH), jnp.int32),
            pltpu.VMEM((2, CH, fc), F32),
            pltpu.VMEM((zc, fc), F32),
            pltpu.VMEM_SHARED((N, fc), F32),
            pltpu.SemaphoreType.DMA,
            pltpu.SemaphoreType.DMA,
        ],
    )
    def k(vals_hbm, dst_hbm, out_hbm, idx_v, v_v, zbuf, acc, sem_d, sem_v):
        cid = lax.axis_index("c")
        sid = lax.axis_index("s")
        for rr in range(zc):
            for j in range(fc // 16):
                zbuf[rr, pl.ds(j * 16, 16)] = jnp.zeros((16,), F32)
        for p in range(fchunks):
            fo = p * fc

            def zbody(cz, _):
                rc = sid + 16 * cz

                @pl.when(rc < nrc)
                def _():
                    pltpu.sync_copy(zbuf, acc.at[pl.ds(rc * zc, zc)])

                return 0

            lax.fori_loop(0, riters, zbody, 0)
            plsc.subcore_barrier()

            def dslc(c):
                chunk = sid + 16 * c
                base = cid * half + chunk * CH
                return (dst_hbm.at[pl.ds(base, CH)],
                        vals_hbm.at[pl.ds(base, CH), pl.ds(fo, fc)], chunk)

            d0, v0, ch0 = dslc(0)

            @pl.when(ch0 < nch)
            def _():
                pltpu.async_copy(d0, idx_v.at[0], sem_d)
                pltpu.async_copy(v0, v_v.at[0], sem_v)

            def body2(j2, _):
                for b in range(2):
                    c = 2 * j2 + b
                    dsl, vsl, chk = dslc(c)
                    pred = chk < nch

                    @pl.when(pred)
                    def _():
                        pltpu.make_async_copy(dsl, idx_v.at[b], sem_d).wait()
                        pltpu.make_async_copy(vsl, v_v.at[b], sem_v).wait()

                    dsn, vsn, chn = dslc(c + 1)

                    @pl.when(chn < nch)
                    def _():
                        pltpu.async_copy(dsn, idx_v.at[1 - b], sem_d)
                        pltpu.async_copy(vsn, v_v.at[1 - b], sem_v)

                    @pl.when(pred)
                    def _():
                        pltpu.sync_copy(v_v.at[b], acc.at[idx_v.at[b]],
                                        add=True)
                return 0

            lax.fori_loop(0, iters // 2, body2, 0)
            plsc.subcore_barrier()

            def wbody(cw, _):
                rc = sid + 16 * cw

                @pl.when(rc < nrc)
                def _():
                    pltpu.sync_copy(
                        acc.at[pl.ds(rc * zc, zc)],
                        out_hbm.at[cid, pl.ds(rc * zc, zc), pl.ds(fo, fc)])

                return 0

            lax.fori_loop(0, riters, wbody, 0)
            plsc.subcore_barrier()

    return k(vals, dst)


def _layer(f, q, src, dst, h, w3, wproj, ng, nb, wq_next, consts):
    sel1, summ, pair, expand = consts
    din = f.shape[1]
    fs = _sc_gather(f, src)
    qd = _sc_gather(q, dst)
    w3all = w3.transpose(2, 0, 1).reshape(din, MID * 2 * CKV)
    lg, v, gmax = _edgec(h, fs, qd, w3all, sel1, summ, pair)
    scv = _edged(lg, v, gmax, expand)
    parts = _sc_scatter_add(scv, dst, C, 1)
    wpz = wproj[:CKV]
    wpf = wproj[CKV:]
    return _nodee(parts, f, wpz, wpf, ng.reshape(1, C), nb.reshape(1, C),
                  expand, wq_next)


def kernel(node_feats, edge_index, edge_w, rel_pos,
           l0_rw1, l0_rb1, l0_rg1, l0_rbt1, l0_rw2, l0_rb2, l0_rg2, l0_rbt2,
           l0_w3, l0_wq, l0_wproj, l0_ng, l0_nb,
           l1_rw1, l1_rb1, l1_rg1, l1_rbt1, l1_rw2, l1_rb2, l1_rg2, l1_rbt2,
           l1_w3, l1_wq, l1_wproj, l1_ng, l1_nb,
           f_rw1, f_rb1, f_rg1, f_rbt1, f_rw2, f_rb2, f_rg2, f_rbt2,
           f_w3, f_wself):
    src = edge_index[0]
    dst = edge_index[1]

    w1 = jnp.stack([l0_rw1, l1_rw1, f_rw1])
    b1 = jnp.stack([l0_rb1, l1_rb1, f_rb1])
    g1 = jnp.stack([l0_rg1, l1_rg1, f_rg1])
    bt1 = jnp.stack([l0_rbt1, l1_rbt1, f_rbt1])
    w2 = jnp.stack([l0_rw2, l1_rw2, f_rw2])
    b2 = jnp.stack([l0_rb2, l1_rb2, f_rb2])
    g2 = jnp.stack([l0_rg2, l1_rg2, f_rg2])
    bt2 = jnp.stack([l0_rbt2, l1_rbt2, f_rbt2])
    h0, h1, hf = _radial(edge_w, rel_pos, w1, b1, g1, bt1, w2, b2, g2, bt2)

    sel1 = jnp.kron(jnp.eye(MID, dtype=F32), jnp.ones((1, 2 * CKV), F32))
    summ = jnp.tile(jnp.eye(2 * CKV, dtype=F32), (MID, 1))
    pair = jnp.kron(jnp.eye(H, dtype=F32), jnp.ones((DH, 1), F32))
    expand = jnp.kron(jnp.eye(H, dtype=F32), jnp.ones((1, DH), F32))
    consts = (sel1, summ, pair, expand)

    q0 = _mm(node_feats, l0_wq)
    f1, q1 = _layer(node_feats, q0, src, dst, h0, l0_w3, l0_wproj,
                    l0_ng, l0_nb, l1_wq, consts)
    f2 = _layer(f1, q1, src, dst, h1, l1_w3, l1_wproj,
                l1_ng, l1_nb, None, consts)[0]

    fsf = _sc_gather(f2, src)
    selh = jnp.kron(jnp.eye(MID, dtype=F32), jnp.ones((1, C), F32))
    self_ = jnp.tile(jnp.eye(C, dtype=F32), (1, MID))
    g = _outer(hf, fsf, selh, self_)
    partsf = _sc_scatter_add(g, dst, 128, 4)
    wf = f_w3.transpose(0, 2, 1).reshape(MID * C, OUT)
    out, pooled = _final(partsf, f2, wf, f_wself)
    return (out[:, :, None], pooled)


# speedup vs baseline: 14.4642x; 1.0732x over previous
import functools
import jax
import jax.numpy as jnp
from jax import lax
from jax.experimental import pallas as pl
from jax.experimental.pallas import tpu as pltpu
from jax.experimental.pallas import tpu_sc as plsc

F32 = jnp.float32

N = 10000
E = 160000
DIN = 128
MID = 16
C = 32
CKV = 16
H = 8
DH = 2
OUT = 1280

BE = 3200
BN = 400
NW = 32
CH = 128


def _ln(x, g, b):
    mu = jnp.mean(x, axis=-1, keepdims=True)
    var = jnp.mean((x - mu) * (x - mu), axis=-1, keepdims=True)
    return (x - mu) * lax.rsqrt(var + 1e-5) * g + b


def _radial_body(ew, rp, w1, b1, g1, bt1, w2, b2, g2, bt2, h0, h1, hf):
    rpv = rp[...]
    r = jnp.sqrt(jnp.sum(rpv * rpv, axis=1, keepdims=True))
    es = jnp.concatenate([ew[...], r], axis=1)
    outs = (h0, h1, hf)
    for j in range(3):
        x = jnp.dot(es, w1[j], preferred_element_type=F32) + b1[j : j + 1]
        x = jax.nn.relu(_ln(x, g1[j : j + 1], bt1[j : j + 1]))
        x = jnp.dot(x, w2[j], preferred_element_type=F32) + b2[j : j + 1]
        x = jax.nn.relu(_ln(x, g2[j : j + 1], bt2[j : j + 1]))
        outs[j][...] = x


def _radial(ew, rp, w1, b1, g1, bt1, w2, b2, g2, bt2):
    nblk = E // BE
    eb = lambda d: pl.BlockSpec((BE, d), lambda i: (i, 0))
    full = lambda s: pl.BlockSpec(s, lambda i: tuple(0 for _ in s))
    return pl.pallas_call(
        _radial_body,
        grid=(nblk,),
        in_specs=[
            eb(4), eb(3),
            full((3, 5, MID)), full((3, MID)), full((3, MID)), full((3, MID)),
            full((3, MID, MID)), full((3, MID)), full((3, MID)), full((3, MID)),
        ],
        out_specs=[eb(MID), eb(MID), eb(MID)],
        out_shape=[jax.ShapeDtypeStruct((E, MID), F32)] * 3,
    )(ew, rp, w1, b1, g1, bt1, w2, b2, g2, bt2)


def _mm_body(x, w, o):
    o[...] = jnp.dot(x[...], w[...], preferred_element_type=F32)


def _mm(x, w):
    n, k = x.shape
    m = w.shape[1]
    return pl.pallas_call(
        _mm_body,
        grid=(n // BN,),
        in_specs=[
            pl.BlockSpec((BN, k), lambda i: (i, 0)),
            pl.BlockSpec((k, m), lambda i: (0, 0)),
        ],
        out_specs=pl.BlockSpec((BN, m), lambda i: (i, 0)),
        out_shape=jax.ShapeDtypeStruct((n, m), F32),
    )(x, w)


def _edgec_body(h, fs, qd, w3all, sel1, summ, pair, lg, v, gmax):
    t = jnp.dot(fs[...], w3all[...], preferred_element_type=F32)
    kvw = t * jnp.dot(h[...], sel1[...], preferred_element_type=F32)
    kv = jnp.dot(kvw, summ[...], preferred_element_type=F32)
    k = kv[:, :CKV]
    vv = kv[:, CKV:]
    prod = k * qd[...]
    lgb = jnp.dot(prod, pair[...], preferred_element_type=F32) * (DH ** -0.5)
    lg[...] = lgb
    v[...] = vv
    bm = jnp.max(lgb, axis=0, keepdims=True)
    i = pl.program_id(0)

    @pl.when(i == 0)
    def _():
        gmax[...] = bm

    @pl.when(i > 0)
    def _():
        gmax[...] = jnp.maximum(gmax[...], bm)


def _edgec(h, fs, qd, w3all, sel1, summ, pair):
    din = fs.shape[1]
    nblk = E // BE
    eb = lambda d: pl.BlockSpec((BE, d), lambda i: (i, 0))
    full = lambda s: pl.BlockSpec(s, lambda i: (0, 0))
    return pl.pallas_call(
        _edgec_body,
        grid=(nblk,),
        in_specs=[
            eb(MID), eb(din), eb(CKV),
            full((din, 2 * CKV * MID)), full((MID, 2 * CKV * MID)),
            full((2 * CKV * MID, 2 * CKV)), full((CKV, H)),
        ],
        out_specs=[eb(H), eb(CKV), full((1, H))],
        out_shape=[
            jax.ShapeDtypeStruct((E, H), F32),
            jax.ShapeDtypeStruct((E, CKV), F32),
            jax.ShapeDtypeStruct((1, H), F32),
        ],
    )(h, fs, qd, w3all, sel1, summ, pair)


def _edged_body(lg, v, gmax, expand, scv):
    ex = jnp.exp(lg[...] - gmax[...])
    num = jnp.dot(ex, expand[...], preferred_element_type=F32) * v[...]
    z8 = jnp.zeros((BE, H), F32)
    scv[...] = jnp.concatenate([ex, num, z8], axis=1)


def _edged(lg, v, gmax, expand):
    nblk = E // BE
    eb = lambda d: pl.BlockSpec((BE, d), lambda i: (i, 0))
    full = lambda s: pl.BlockSpec(s, lambda i: (0, 0))
    return pl.pallas_call(
        _edged_body,
        grid=(nblk,),
        in_specs=[eb(H), eb(CKV), full((1, H)), full((H, CKV))],
        out_specs=eb(C),
        out_shape=jax.ShapeDtypeStruct((E, C), F32),
    )(lg, v, gmax, expand)


def _nodee_body(parts, f, wpz, wpf, ng, nb, expand, wqn, fout, qout):
    p = parts[...]
    s = p[0] + p[1]
    den = s[:, :H]
    num = s[:, H : H + CKV]
    dexp = jnp.dot(den, expand[...], preferred_element_type=F32) + 1e-30
    z = num / dexp
    fp = jnp.dot(z, wpz[...], preferred_element_type=F32) + jnp.dot(
        f[...], wpf[...], preferred_element_type=F32)
    nf = jnp.abs(fp)
    nn = jax.nn.relu(_ln(nf, ng[...], nb[...]))
    fnew = fp / (nf + 1e-8) * nn
    fout[...] = fnew
    if qout is not None:
        qout[...] = jnp.dot(fnew, wqn[...], preferred_element_type=F32)


def _nodee(parts, f, wpz, wpf, ng, nb, expand, wqn):
    din = f.shape[1]
    full = lambda s: pl.BlockSpec(s, lambda i: tuple(0 for _ in s))
    in_specs = [
        pl.BlockSpec((2, BN, C), lambda i: (0, i, 0)),
        pl.BlockSpec((BN, din), lambda i: (i, 0)),
        full((CKV, C)), full((din, C)), full((1, C)), full((1, C)),
        full((H, CKV)),
    ]
    out_specs = [pl.BlockSpec((BN, C), lambda i: (i, 0))]
    out_shape = [jax.ShapeDtypeStruct((N, C), F32)]
    if wqn is not None:
        in_specs.append(full((C, CKV)))
        out_specs.append(pl.BlockSpec((BN, CKV), lambda i: (i, 0)))
        out_shape.append(jax.ShapeDtypeStruct((N, CKV), F32))
        body = _nodee_body
        args = (parts, f, wpz, wpf, ng, nb, expand, wqn)
    else:
        def body(parts, f, wpz, wpf, ng, nb, expand, fout):
            _nodee_body(parts, f, wpz, wpf, ng, nb, expand, None, fout, None)
        args = (parts, f, wpz, wpf, ng, nb, expand)
    return pl.pallas_call(
        body,
        grid=(N // BN,),
        in_specs=in_specs,
        out_specs=out_specs,
        out_shape=out_shape,
    )(*args)


def _outer_body(h, fs, selh, self_, g):
    g[...] = jnp.dot(h[...], selh[...], preferred_element_type=F32) * jnp.dot(
        fs[...], self_[...], preferred_element_type=F32)


def _outer(h, fs, selh, self_):
    nblk = E // BE
    eb = lambda d: pl.BlockSpec((BE, d), lambda i: (i, 0))
    full = lambda s: pl.BlockSpec(s, lambda i: (0, 0))
    return pl.pallas_call(
        _outer_body,
        grid=(nblk,),
        in_specs=[eb(MID), eb(C), full((MID, MID * C)), full((C, MID * C))],
        out_specs=eb(MID * C),
        out_shape=jax.ShapeDtypeStruct((E, MID * C), F32),
    )(h, fs, selh, self_)


def _final_body(parts, f, wf, wself, out, pooled):
    p = parts[...]
    a = p[0] + p[1]
    o = jnp.dot(a, wf[...], preferred_element_type=F32) + jnp.dot(
        f[...], wself[...], preferred_element_type=F32)
    out[...] = o
    bm = jnp.max(o, axis=0, keepdims=True)
    i = pl.program_id(0)

    @pl.when(i == 0)
    def _():
        pooled[...] = bm

    @pl.when(i > 0)
    def _():
        pooled[...] = jnp.maximum(pooled[...], bm)


def _final(parts, f, wf, wself):
    full = lambda s: pl.BlockSpec(s, lambda i: (0, 0))
    return pl.pallas_call(
        _final_body,
        grid=(N // BN,),
        in_specs=[
            pl.BlockSpec((2, BN, MID * C), lambda i: (0, i, 0)),
            pl.BlockSpec((BN, C), lambda i: (i, 0)),
            full((MID * C, OUT)), full((C, OUT)),
        ],
        out_specs=[pl.BlockSpec((BN, OUT), lambda i: (i, 0)), full((1, OUT))],
        out_shape=[
            jax.ShapeDtypeStruct((N, OUT), F32),
            jax.ShapeDtypeStruct((1, OUT), F32),
        ],
    )(parts, f, wf, wself)


def _sc_gather(table, idx):
    d = table.shape[1]
    e = idx.shape[0]
    nch = e // CH
    iters = (nch + NW - 1) // NW
    tiled = d % 128 == 0
    mesh = plsc.VectorSubcoreMesh(core_axis_name="c", subcore_axis_name="s")

    @functools.partial(
        pl.kernel,
        out_type=jax.ShapeDtypeStruct((e, d), F32),
        mesh=mesh,
        compiler_params=pltpu.CompilerParams(use_tc_tiling_on_sc=tiled),
        scratch_types=[
            pltpu.VMEM((2, CH), jnp.int32),
            pltpu.VMEM((2, CH, d), F32),
            pltpu.SemaphoreType.DMA,
            pltpu.SemaphoreType.DMA,
            pltpu.SemaphoreType.DMA,
            pltpu.SemaphoreType.DMA,
        ],
    )
    def k(table_hbm, idx_hbm, out_hbm, idx_v, rows_v, sem_i, sem_g, so0, so1):
        wid = lax.axis_index("s") * 2 + lax.axis_index("c")
        sem_o = (so0, so1)

        def islc(c):
            base = jnp.minimum(wid + NW * c, nch - 1) * CH
            return idx_hbm.at[pl.ds(base, CH)], out_hbm.at[pl.ds(base, CH)]

        i0, _o0 = islc(0)
        pltpu.async_copy(i0, idx_v.at[0], sem_i)

        def body2(j2, _):
            for b in range(2):
                c = 2 * j2 + b
                isl, osl = islc(c)
                pltpu.make_async_copy(isl, idx_v.at[b], sem_i).wait()
                isn, _osn = islc(c + 1)

                @pl.when(c + 1 < iters)
                def _():
                    pltpu.async_copy(isn, idx_v.at[1 - b], sem_i)

                @pl.when(c >= 2)
                def _():
                    _ipre, opre = islc(c - 2)
                    pltpu.make_async_copy(rows_v.at[b], opre, sem_o[b]).wait()

                pltpu.async_copy(
                    table_hbm.at[idx_v.at[b]], rows_v.at[b], sem_g).wait()
                pltpu.async_copy(rows_v.at[b], osl, sem_o[b])
            return 0

        lax.fori_loop(0, iters // 2, body2, 0)
        for c in (iters - 2, iters - 1):
            b = c % 2
            _il, ol = islc(c)
            pltpu.make_async_copy(rows_v.at[b], ol, sem_o[b]).wait()

    return k(table, idx)


def _sc_scatter_add(vals, dst, fc, fchunks):
    e = dst.shape[0]
    half = e // 2
    nch = half // CH
    iters = (nch + 15) // 16
    zc = 40
    nrc = N // zc
    riters = (nrc + 15) // 16
    ftot = vals.shape[1]
    tiled = fc % 128 == 0
    mesh = plsc.VectorSubcoreMesh(core_axis_name="c", subcore_axis_name="s")

    @functools.partial(
        pl.kernel,
        out_type=jax.ShapeDtypeStruct((2, N, ftot), F32),
        mesh=mesh,
        compiler_params=pltpu.CompilerParams(use_tc_tiling_on_sc=tiled),
        scratch_types=[
            pltpu.VMEM((2, CH), jnp.int32),
            pltpu.VMEM((2, CH, fc), F32),
            pltpu.VMEM((zc, fc), F32),
            pltpu.VMEM_SHARED((N, fc), F32),
            pltpu.SemaphoreType.DMA,
            pltpu.SemaphoreType.DMA,
        ],
    )
    def k(vals_hbm, dst_hbm, out_hbm, idx_v, v_v, zbuf, acc, sem_d, sem_v):
        cid = lax.axis_index("c")
        sid = lax.axis_index("s")
        for rr in range(zc):
            for j in range(fc // 16):
                zbuf[rr, pl.ds(j * 16, 16)] = jnp.zeros((16,), F32)
        for p in range(fchunks):
            fo = p * fc

            def zbody(cz, _):
                rc = sid + 16 * cz

                @pl.when(rc < nrc)
                def _():
                    pltpu.sync_copy(zbuf, acc.at[pl.ds(rc * zc, zc)])

                return 0

            lax.fori_loop(0, riters, zbody, 0)
            plsc.subcore_barrier()

            def dslc(c):
                chunk = sid + 16 * c
                base = cid * half + chunk * CH
                return (dst_hbm.at[pl.ds(base, CH)],
                        vals_hbm.at[pl.ds(base, CH), pl.ds(fo, fc)], chunk)

            d0, v0, ch0 = dslc(0)

            @pl.when(ch0 < nch)
            def _():
                pltpu.async_copy(d0, idx_v.at[0], sem_d)
                pltpu.async_copy(v0, v_v.at[0], sem_v)

            def body2(j2, _):
                for b in range(2):
                    c = 2 * j2 + b
                    dsl, vsl, chk = dslc(c)
                    pred = chk < nch

                    @pl.when(pred)
                    def _():
                        pltpu.make_async_copy(dsl, idx_v.at[b], sem_d).wait()
                        pltpu.make_async_copy(vsl, v_v.at[b], sem_v).wait()

                    dsn, vsn, chn = dslc(c + 1)

                    @pl.when(chn < nch)
                    def _():
                        pltpu.async_copy(dsn, idx_v.at[1 - b], sem_d)
                        pltpu.async_copy(vsn, v_v.at[1 - b], sem_v)

                    @pl.when(pred)
                    def _():
                        pltpu.sync_copy(v_v.at[b], acc.at[idx_v.at[b]],
                                        add=True)
                return 0

            lax.fori_loop(0, iters // 2, body2, 0)
            plsc.subcore_barrier()

            def wbody(cw, _):
                rc = sid + 16 * cw

                @pl.when(rc < nrc)
                def _():
                    pltpu.sync_copy(
                        acc.at[pl.ds(rc * zc, zc)],
                        out_hbm.at[cid, pl.ds(rc * zc, zc), pl.ds(fo, fc)])

                return 0

            lax.fori_loop(0, riters, wbody, 0)
            plsc.subcore_barrier()

    return k(vals, dst)


def _layer(f, q, src, dst, h, w3, wproj, ng, nb, wq_next, consts):
    sel1, summ, pair, expand = consts
    din = f.shape[1]
    fs = _sc_gather(f, src)
    qd = _sc_gather(q, dst)
    w3all = w3.transpose(2, 0, 1).reshape(din, MID * 2 * CKV)
    lg, v, gmax = _edgec(h, fs, qd, w3all, sel1, summ, pair)
    scv = _edged(lg, v, gmax, expand)
    parts = _sc_scatter_add(scv, dst, C, 1)
    wpz = wproj[:CKV]
    wpf = wproj[CKV:]
    return _nodee(parts, f, wpz, wpf, ng.reshape(1, C), nb.reshape(1, C),
                  expand, wq_next)


def kernel(node_feats, edge_index, edge_w, rel_pos,
           l0_rw1, l0_rb1, l0_rg1, l0_rbt1, l0_rw2, l0_rb2, l0_rg2, l0_rbt2,
           l0_w3, l0_wq, l0_wproj, l0_ng, l0_nb,
           l1_rw1, l1_rb1, l1_rg1, l1_rbt1, l1_rw2, l1_rb2, l1_rg2, l1_rbt2,
           l1_w3, l1_wq, l1_wproj, l1_ng, l1_nb,
           f_rw1, f_rb1, f_rg1, f_rbt1, f_rw2, f_rb2, f_rg2, f_rbt2,
           f_w3, f_wself):
    src = edge_index[0]
    dst = edge_index[1]

    w1 = jnp.stack([l0_rw1, l1_rw1, f_rw1])
    b1 = jnp.stack([l0_rb1, l1_rb1, f_rb1])
    g1 = jnp.stack([l0_rg1, l1_rg1, f_rg1])
    bt1 = jnp.stack([l0_rbt1, l1_rbt1, f_rbt1])
    w2 = jnp.stack([l0_rw2, l1_rw2, f_rw2])
    b2 = jnp.stack([l0_rb2, l1_rb2, f_rb2])
    g2 = jnp.stack([l0_rg2, l1_rg2, f_rg2])
    bt2 = jnp.stack([l0_rbt2, l1_rbt2, f_rbt2])
    h0, h1, hf = _radial(edge_w, rel_pos, w1, b1, g1, bt1, w2, b2, g2, bt2)

    sel1 = jnp.kron(jnp.eye(MID, dtype=F32), jnp.ones((1, 2 * CKV), F32))
    summ = jnp.tile(jnp.eye(2 * CKV, dtype=F32), (MID, 1))
    pair = jnp.kron(jnp.eye(H, dtype=F32), jnp.ones((DH, 1), F32))
    expand = jnp.kron(jnp.eye(H, dtype=F32), jnp.ones((1, DH), F32))
    consts = (sel1, summ, pair, expand)

    q0 = _mm(node_feats, l0_wq)
    f1, q1 = _layer(node_feats, q0, src, dst, h0, l0_w3, l0_wproj,
                    l0_ng, l0_nb, l1_wq, consts)
    f2 = _layer(f1, q1, src, dst, h1, l1_w3, l1_wproj,
                l1_ng, l1_nb, None, consts)[0]

    fsf = _sc_gather(f2, src)
    selh = jnp.kron(jnp.eye(MID, dtype=F32), jnp.ones((1, C), F32))
    self_ = jnp.tile(jnp.eye(C, dtype=F32), (1, MID))
    g = _outer(hf, fsf, selh, self_)
    partsf = _sc_scatter_add(g, dst, 128, 4)
    wf = f_w3.transpose(0, 2, 1).reshape(MID * C, OUT)
    out, pooled = _final(partsf, f2, wf, f_wself)
    return (out[:, :, None], pooled)
